# Initial kernel scaffold; baseline (speedup 1.0000x reference)
import jax
import jax.numpy as jnp
from jax.experimental import pallas as pl


def kernel(x, edge_index, batch, params):
    raise NotImplementedError("write your pallas kernel here")


# speedup vs baseline: 24.9424x; 24.9424x over previous
import functools

import jax
import jax.numpy as jnp
import numpy as np
from jax import lax
from jax.experimental import pallas as pl
from jax.experimental.pallas import tpu as pltpu
from jax.experimental.pallas import tpu_sc as plsc

N_NODES = 10000
N_PAD = 10240
N_EDGES = 320000
D = 128
HEADS = 4
CH = 32
N_GRAPHS = 16

NC = 2
NS = 16
NW = NC * NS
K = 80
EPT = N_EDGES // NW
NCHUNK = EPT // K
ROWS_PER_TILE = N_PAD // NS

_f32 = jnp.float32
_i32 = jnp.int32


def _sc_gatex_body(asrc_hbm, adst_hbm, src_hbm, dst_hbm, ex_hbm,
                   src_v, dst_v, ex_v, asrc_t, adst_t):
    cid = lax.axis_index("c")
    sid = lax.axis_index("s")
    wid = sid * NC + cid
    pltpu.sync_copy(asrc_hbm, asrc_t)
    pltpu.sync_copy(adst_hbm, adst_t)

    lanes = lax.iota(_i32, 16)
    ones16 = jnp.ones((16,), _f32)

    def chunk(i, carry):
        eoff = wid * EPT + i * K
        pltpu.sync_copy(src_hbm.at[pl.ds(eoff, K)], src_v)
        pltpu.sync_copy(dst_hbm.at[pl.ds(eoff, K)], dst_v)
        for g in range(K // 16):
            ridx = lanes + (g * 16)
            sids = src_v[pl.ds(g * 16, 16)] * 4
            dids = dst_v[pl.ds(g * 16, 16)] * 4
            for hh in range(HEADS):
                s_ = plsc.load_gather(asrc_t, [sids + hh])
                d_ = plsc.load_gather(adst_t, [dids + hh])
                al = s_ + d_
                al = jnp.where(al > 0, al, 0.2 * al)
                plsc.store_scatter(ex_v, [ridx, jnp.full((16,), hh, _i32)],
                                   jnp.exp(al))
            plsc.store_scatter(ex_v, [ridx, jnp.full((16,), HEADS, _i32)], ones16)
        pltpu.sync_copy(ex_v, ex_hbm.at[pl.ds(eoff, K)])
        return carry

    lax.fori_loop(0, NCHUNK, chunk, 0)


DEN_ROWS = N_PAD // 8
DEN_TILE = DEN_ROWS // NS


def _sc_gat_body(h_hbm, ex_hbm, src_hbm, dst_hbm, zeros_hbm,
                 num_hbm, den_hbm,
                 src_v, dst_v, drow_v, rows_v, ex_v, exw_v, acc, accden):
    cid = lax.axis_index("c")
    sid = lax.axis_index("s")
    wid = sid * NC + cid
    r0 = sid * ROWS_PER_TILE
    pltpu.sync_copy(zeros_hbm, acc.at[pl.ds(r0, ROWS_PER_TILE)])
    pltpu.sync_copy(zeros_hbm.at[pl.ds(0, DEN_TILE)],
                    accden.at[pl.ds(sid * DEN_TILE, DEN_TILE)])
    pltpu.sync_copy(zeros_hbm.at[pl.ds(0, K)], exw_v)
    plsc.subcore_barrier()

    lanes = lax.iota(_i32, 16)
    zeros16v = jnp.zeros((16,), _f32)

    def chunk(i, carry):
        eoff = wid * EPT + i * K
        pltpu.sync_copy(src_hbm.at[pl.ds(eoff, K)], src_v)
        pltpu.sync_copy(dst_hbm.at[pl.ds(eoff, K)], dst_v)
        pltpu.sync_copy(ex_hbm.at[pl.ds(eoff, K)], ex_v)
        pltpu.sync_copy(h_hbm.at[src_v], rows_v)
        for g in range(K // 16):
            ridx = lanes + (g * 16)
            dvals = dst_v[pl.ds(g * 16, 16)]
            drow_v[pl.ds(g * 16, 16)] = lax.shift_right_logical(dvals, 3)
            cbase = (dvals & 7) * 16
            for hh in range(HEADS + 1):
                vals = plsc.load_gather(ex_v, [ridx, jnp.full((16,), hh, _i32)])
                plsc.store_scatter(exw_v, [ridx, cbase + hh], vals)
            for e in range(16):
                row = g * 16 + e
                for hh in range(HEADS):
                    w = plsc.load_gather(
                        ex_v, [jnp.full((16,), row, _i32),
                               jnp.full((16,), hh, _i32)])
                    for d2 in range(2):
                        c0 = (hh * 2 + d2) * 16
                        rows_v[row, pl.ds(c0, 16)] = rows_v[row, pl.ds(c0, 16)] * w
        pltpu.sync_copy(rows_v, acc.at[dst_v], add=True)
        pltpu.sync_copy(exw_v, accden.at[drow_v], add=True)
        for g in range(K // 16):
            ridx = lanes + (g * 16)
            dvals = dst_v[pl.ds(g * 16, 16)]
            cbase = (dvals & 7) * 16
            for hh in range(HEADS + 1):
                plsc.store_scatter(exw_v, [ridx, cbase + hh], zeros16v)
        return carry

    lax.fori_loop(0, NCHUNK, chunk, 0)
    plsc.subcore_barrier()
    pltpu.sync_copy(acc.at[pl.ds(r0, ROWS_PER_TILE)],
                    num_hbm.at[cid, pl.ds(r0, ROWS_PER_TILE)])
    pltpu.sync_copy(accden.at[pl.ds(sid * DEN_TILE, DEN_TILE)],
                    den_hbm.at[cid, pl.ds(sid * DEN_TILE, DEN_TILE)])


def _sc_sage_body(h_hbm, src_hbm, dst_hbm, zeros_hbm, s_hbm,
                  src_v, dst_v, rows_v, acc):
    cid = lax.axis_index("c")
    sid = lax.axis_index("s")
    wid = sid * NC + cid
    r0 = sid * ROWS_PER_TILE
    pltpu.sync_copy(zeros_hbm, acc.at[pl.ds(r0, ROWS_PER_TILE)])
    plsc.subcore_barrier()

    def chunk(i, carry):
        eoff = wid * EPT + i * K
        pltpu.sync_copy(src_hbm.at[pl.ds(eoff, K)], src_v)
        pltpu.sync_copy(dst_hbm.at[pl.ds(eoff, K)], dst_v)
        pltpu.sync_copy(h_hbm.at[src_v], rows_v)
        pltpu.sync_copy(rows_v, acc.at[dst_v], add=True)
        return carry

    lax.fori_loop(0, NCHUNK, chunk, 0)
    plsc.subcore_barrier()
    pltpu.sync_copy(acc.at[pl.ds(r0, ROWS_PER_TILE)],
                    s_hbm.at[cid, pl.ds(r0, ROWS_PER_TILE)])


def _make_sc_mesh():
    return plsc.VectorSubcoreMesh(core_axis_name="c", subcore_axis_name="s")


_SC_PARAMS = pltpu.CompilerParams(needs_layout_passes=False)


def _sc_gatex(asrc_flat, adst_flat, src, dst):
    return pl.kernel(
        _sc_gatex_body,
        out_type=jax.ShapeDtypeStruct((N_EDGES, 16), _f32),
        mesh=_make_sc_mesh(),
        scratch_types=[
            pltpu.VMEM((K,), _i32),
            pltpu.VMEM((K,), _i32),
            pltpu.VMEM((K, 16), _f32),
            pltpu.VMEM((HEADS * N_NODES,), _f32),
            pltpu.VMEM((HEADS * N_NODES,), _f32),
        ],
        compiler_params=_SC_PARAMS,
    )(asrc_flat, adst_flat, src, dst)


def _sc_gat(h, ex, src, dst, zeros):
    return pl.kernel(
        _sc_gat_body,
        out_type=[
            jax.ShapeDtypeStruct((NC, N_PAD, D), _f32),
            jax.ShapeDtypeStruct((NC, DEN_ROWS, D), _f32),
        ],
        mesh=_make_sc_mesh(),
        scratch_types=[
            pltpu.VMEM((K,), _i32),
            pltpu.VMEM((K,), _i32),
            pltpu.VMEM((K,), _i32),
            pltpu.VMEM((K, D), _f32),
            pltpu.VMEM((K, 16), _f32),
            pltpu.VMEM((K, D), _f32),
            pltpu.VMEM_SHARED((N_PAD, D), _f32),
            pltpu.VMEM_SHARED((DEN_ROWS, D), _f32),
        ],
        compiler_params=_SC_PARAMS,
    )(h, ex, src, dst, zeros)


def _sc_sage(h, src, dst, zeros):
    return pl.kernel(
        _sc_sage_body,
        out_type=jax.ShapeDtypeStruct((NC, N_PAD, D), _f32),
        mesh=_make_sc_mesh(),
        scratch_types=[
            pltpu.VMEM((K,), _i32),
            pltpu.VMEM((K,), _i32),
            pltpu.VMEM((K, D), _f32),
            pltpu.VMEM_SHARED((N_PAD, D), _f32),
        ],
        compiler_params=_SC_PARAMS,
    )(h, src, dst, zeros)


_HI = dict(preferred_element_type=_f32, precision=lax.Precision.HIGHEST)
BROW = 2000
GRID = N_NODES // BROW


def _full(shape):
    return pl.BlockSpec(shape, lambda i: (0,) * len(shape))


def _rows(minor):
    return pl.BlockSpec((BROW, minor), lambda i: (i, 0))


def _prows(minor):
    return pl.BlockSpec((NC, BROW, minor), lambda i: (0, i, 0))


def _tc_pre_body(x_ref, w_ref, ms_ref, md_ref, wr_ref, br_ref,
                 h_ref, as_ref, ad_ref, res_ref):
    x = x_ref[...]
    h = jnp.dot(x, w_ref[...], **_HI)
    h_ref[...] = h
    as_ref[...] = jnp.dot(h, ms_ref[...], **_HI)
    ad_ref[...] = jnp.dot(h, md_ref[...], **_HI)
    res_ref[...] = jnp.dot(x, wr_ref[...], **_HI) + br_ref[...]


def _tc_pre(x, w, ms, md, wr, br):
    return pl.pallas_call(
        _tc_pre_body,
        grid=(GRID,),
        in_specs=[_rows(D), _full((D, D)), _full((D, HEADS)),
                  _full((D, HEADS)), _full((D, D)), _full((1, D))],
        out_specs=[_rows(D), _rows(HEADS), _rows(HEADS), _rows(D)],
        out_shape=[
            jax.ShapeDtypeStruct((N_NODES, D), _f32),
            jax.ShapeDtypeStruct((N_NODES, HEADS), _f32),
            jax.ShapeDtypeStruct((N_NODES, HEADS), _f32),
            jax.ShapeDtypeStruct((N_NODES, D), _f32),
        ],
    )(x, w, ms, md, wr, br)


def _bn_lrelu(out, g, b, res):
    m = jnp.mean(out, axis=0, keepdims=True)
    v = jnp.mean((out - m) * (out - m), axis=0, keepdims=True)
    out = (out - m) / jnp.sqrt(v + 1e-5) * g + b
    out = out + res
    return jnp.where(out > 0, out, 0.2 * out)


def _tc_bnres_body(o_ref, g_ref, bb_ref, res_ref, hf_ref):
    hf_ref[...] = _bn_lrelu(o_ref[...], g_ref[...], bb_ref[...], res_ref[...])


def _tc_bnres(out, g, bb, res):
    return pl.pallas_call(
        _tc_bnres_body,
        out_shape=jax.ShapeDtypeStruct((N_NODES, D), _f32),
    )(out, g, bb, res)


def _tc_bn_body(o_ref, g_ref, bb_ref, hf_ref):
    hf_ref[...] = _bn_lrelu(o_ref[...], g_ref[...], bb_ref[...], 0.0)


def _tc_bn(out, g, bb):
    return pl.pallas_call(
        _tc_bn_body,
        out_shape=jax.ShapeDtypeStruct((N_NODES, D), _f32),
    )(out, g, bb)


def _tc_gatcomb_body(nump, denp, h_ref, as_ref, ad_ref, e_ref, eh_ref,
                     b_ref, o_ref, den_ref):
    num = nump[0] + nump[1]
    den16 = denp[0] + denp[1]
    aself = as_ref[...] + ad_ref[...]
    ex4 = jnp.exp(jnp.where(aself > 0, aself, 0.2 * aself))
    exx = jnp.dot(ex4, eh_ref[...], **_HI)
    numt = num + h_ref[...] * exx
    denx = jnp.dot(den16, e_ref[...], **_HI) + exx
    o_ref[...] = numt / (denx + 1e-16) + b_ref[...]
    den_ref[...] = den16


def _tc_gatcomb(nump, denp, h, as4, ad4, e, eh, b):
    return pl.pallas_call(
        _tc_gatcomb_body,
        grid=(GRID,),
        in_specs=[_prows(D), _prows(16), _rows(D), _rows(HEADS), _rows(HEADS),
                  _full((16, D)), _full((HEADS, D)), _full((1, D))],
        out_specs=[_rows(D), _rows(16)],
        out_shape=[
            jax.ShapeDtypeStruct((N_NODES, D), _f32),
            jax.ShapeDtypeStruct((N_NODES, 16), _f32),
        ],
    )(nump, denp, h, as4, ad4, e, eh, b)


def _tc_sagecomb_body(sp, den_ref, e4_ref, hin_ref, wl_ref, bl_ref, wr_ref,
                      o_ref):
    s = sp[0] + sp[1]
    cntx = jnp.dot(den_ref[...], e4_ref[...], **_HI)
    mean = s / jnp.maximum(cntx, 1.0)
    o_ref[...] = (jnp.dot(mean, wl_ref[...], **_HI) + bl_ref[...]
                  + jnp.dot(hin_ref[...], wr_ref[...], **_HI))


def _tc_sagecomb(sp, den16, e4, hin, wl, bl, wr):
    return pl.pallas_call(
        _tc_sagecomb_body,
        grid=(GRID,),
        in_specs=[_prows(D), _rows(16), _full((16, D)), _rows(D),
                  _full((D, D)), _full((1, D)), _full((D, D))],
        out_specs=_rows(D),
        out_shape=jax.ShapeDtypeStruct((N_NODES, D), _f32),
    )(sp, den16, e4, hin, wl, bl, wr)


def _tc_pool_body(h_ref, batch_ref, sums_ref, cnt_ref):
    i = pl.program_id(0)
    bt = batch_ref[...]
    oh = (bt == lax.broadcasted_iota(_i32, (1, N_GRAPHS), 1)).astype(_f32)
    part = lax.dot_general(oh, h_ref[...], (((0,), (0,)), ((), ())), **_HI)
    ones = jnp.ones((BROW, N_GRAPHS), _f32)
    pcnt = lax.dot_general(oh, ones, (((0,), (0,)), ((), ())),
                           preferred_element_type=_f32)

    @pl.when(i == 0)
    def _():
        sums_ref[...] = jnp.zeros((N_GRAPHS, D), _f32)
        cnt_ref[...] = jnp.zeros((N_GRAPHS, N_GRAPHS), _f32)

    sums_ref[...] += part
    cnt_ref[...] += pcnt


def _tc_pool(h5, batch2d):
    return pl.pallas_call(
        _tc_pool_body,
        grid=(GRID,),
        in_specs=[_rows(D), _rows(1)],
        out_specs=[pl.BlockSpec((N_GRAPHS, D), lambda i: (0, 0)),
                   pl.BlockSpec((N_GRAPHS, N_GRAPHS), lambda i: (0, 0))],
        out_shape=[
            jax.ShapeDtypeStruct((N_GRAPHS, D), _f32),
            jax.ShapeDtypeStruct((N_GRAPHS, N_GRAPHS), _f32),
        ],
    )(h5, batch2d)


def _tc_fc_body(sums_ref, cnt_ref, w_ref, b_ref, out_ref):
    cnt = cnt_ref[:, :1]
    gm = sums_ref[...] / jnp.maximum(cnt, 1.0)
    out_ref[...] = jnp.dot(gm, w_ref[...], **_HI) + b_ref[...]


def _tc_fc(sums, cnt, w, b):
    return pl.pallas_call(
        _tc_fc_body,
        out_shape=jax.ShapeDtypeStruct((N_GRAPHS, w.shape[1]), _f32),
    )(sums, cnt, w, b)


def _att_mat(att):
    a = att.reshape(HEADS, CH)
    eye = jnp.eye(HEADS, dtype=_f32)
    return jnp.einsum('hc,hk->hck', a, eye).reshape(D, HEADS)


_E16 = np.zeros((16, D), np.float32)
for _h in range(HEADS):
    _E16[_h, _h * CH:(_h + 1) * CH] = 1.0

_EH = np.zeros((HEADS, D), np.float32)
for _h in range(HEADS):
    _EH[_h, _h * CH:(_h + 1) * CH] = 1.0

_E4 = np.zeros((16, D), np.float32)
_E4[HEADS, :] = 1.0


def kernel(x, edge_index, batch, params):
    p = params
    src = edge_index[0]
    dst = edge_index[1]
    zeros = jnp.zeros((ROWS_PER_TILE, D), _f32)
    zeros16 = jnp.zeros((ROWS_PER_TILE, 16), _f32)
    e16 = jnp.asarray(_E16)
    eh = jnp.asarray(_EH)
    e4 = jnp.asarray(_E4)

    def row(v):
        return v.reshape(1, -1)

    h1, as1, ad1, res1 = _tc_pre(x, p['gat1']['W'],
                                 _att_mat(p['gat1']['att_src']),
                                 _att_mat(p['gat1']['att_dst']),
                                 p['res1']['W'], row(p['res1']['b']))
    ex1 = _sc_gatex(as1.reshape(-1), ad1.reshape(-1), src, dst)
    num1, den1 = _sc_gat(h1, ex1, src, dst, zeros)
    den1 = den1.reshape(NC, N_PAD, 16)
    o1, den16 = _tc_gatcomb(num1, den1, h1, as1, ad1, e16, eh,
                            row(p['gat1']['b']))
    h1f = _tc_bnres(o1, row(p['bn1']['g']), row(p['bn1']['b']), res1)

    h2, as2, ad2, res2 = _tc_pre(h1f, p['gat2']['W'],
                                 _att_mat(p['gat2']['att_src']),
                                 _att_mat(p['gat2']['att_dst']),
                                 p['res2']['W'], row(p['res2']['b']))
    ex2 = _sc_gatex(as2.reshape(-1), ad2.reshape(-1), src, dst)
    num2, den2 = _sc_gat(h2, ex2, src, dst, zeros)
    den2 = den2.reshape(NC, N_PAD, 16)
    o2, _ = _tc_gatcomb(num2, den2, h2, as2, ad2, e16, eh,
                        row(p['gat2']['b']))
    h2f = _tc_bnres(o2, row(p['bn2']['g']), row(p['bn2']['b']), res2)

    h = h2f
    for name, bn in (('sage3', 'bn3'), ('sage4', 'bn4'), ('sage5', 'bn5')):
        sp = _sc_sage(h, src, dst, zeros)
        o = _tc_sagecomb(sp, den16, e4, h,
                         p[name]['Wl'], row(p[name]['bl']), p[name]['Wr'])
        h = _tc_bn(o, row(p[bn]['g']), row(p[bn]['b']))

    sums, cnt = _tc_pool(h, batch.reshape(-1, 1))
    return _tc_fc(sums, cnt, p['fc']['W'], row(p['fc']['b']))


# speedup vs baseline: 33.5346x; 1.3445x over previous
import functools

import jax
import jax.numpy as jnp
import numpy as np
from jax import lax
from jax.experimental import pallas as pl
from jax.experimental.pallas import tpu as pltpu
from jax.experimental.pallas import tpu_sc as plsc

N_NODES = 10000
N_PAD = 10240
N_EDGES = 320000
D = 128
HEADS = 4
CH = 32
N_GRAPHS = 16

NC = 2
NS = 16
NW = NC * NS
K = 80
EPT = N_EDGES // NW
NCHUNK = EPT // K
ROWS_PER_TILE = N_PAD // NS

_f32 = jnp.float32
_i32 = jnp.int32


DEN_ROWS = N_PAD // 8
DEN_TILE = DEN_ROWS // NS


def _sc_gatex_body(asrc_hbm, adst_hbm, src_hbm, dst_hbm, zeros_hbm,
                   ex_hbm, den_hbm,
                   src_v, dst_v, drow_v, ex_v, exw_v, asrc_t, adst_t, accden):
    cid = lax.axis_index("c")
    sid = lax.axis_index("s")
    wid = sid * NC + cid
    pltpu.sync_copy(asrc_hbm, asrc_t)
    pltpu.sync_copy(adst_hbm, adst_t)
    pltpu.sync_copy(zeros_hbm.at[pl.ds(0, DEN_TILE)],
                    accden.at[pl.ds(sid * DEN_TILE, DEN_TILE)])
    pltpu.sync_copy(zeros_hbm.at[pl.ds(0, K)], exw_v)
    plsc.subcore_barrier()

    lanes = lax.iota(_i32, 16)
    ones16 = jnp.ones((16,), _f32)
    zeros16v = jnp.zeros((16,), _f32)

    def chunk(i, carry):
        eoff = wid * EPT + i * K
        pltpu.sync_copy(src_hbm.at[pl.ds(eoff, K)], src_v)
        pltpu.sync_copy(dst_hbm.at[pl.ds(eoff, K)], dst_v)
        for g in range(K // 16):
            ridx = lanes + (g * 16)
            sids = src_v[pl.ds(g * 16, 16)] * 4
            dvals = dst_v[pl.ds(g * 16, 16)]
            dids = dvals * 4
            drow_v[pl.ds(g * 16, 16)] = lax.shift_right_logical(dvals, 3)
            cbase = (dvals & 7) * 16
            for hh in range(HEADS):
                s_ = plsc.load_gather(asrc_t, [sids + hh])
                d_ = plsc.load_gather(adst_t, [dids + hh])
                al = s_ + d_
                al = jnp.where(al > 0, al, 0.2 * al)
                exv = jnp.exp(al)
                plsc.store_scatter(ex_v, [ridx, jnp.full((16,), hh, _i32)], exv)
                plsc.store_scatter(exw_v, [ridx, cbase + hh], exv)
            plsc.store_scatter(ex_v, [ridx, jnp.full((16,), HEADS, _i32)], ones16)
            plsc.store_scatter(exw_v, [ridx, cbase + HEADS], ones16)
        pltpu.sync_copy(ex_v, ex_hbm.at[pl.ds(eoff, K)])
        pltpu.sync_copy(exw_v, accden.at[drow_v], add=True)
        for g in range(K // 16):
            ridx = lanes + (g * 16)
            cbase = (dst_v[pl.ds(g * 16, 16)] & 7) * 16
            for hh in range(HEADS + 1):
                plsc.store_scatter(exw_v, [ridx, cbase + hh], zeros16v)
        return carry

    lax.fori_loop(0, NCHUNK, chunk, 0)
    plsc.subcore_barrier()
    pltpu.sync_copy(accden.at[pl.ds(sid * DEN_TILE, DEN_TILE)],
                    den_hbm.at[cid, pl.ds(sid * DEN_TILE, DEN_TILE)])


def _sc_gat_body(h_hbm, ex_hbm, src_hbm, dst_hbm, zeros_hbm, num_hbm,
                 src0, src1, dst0, dst1, rows0, rows1, ex0, ex1, acc,
                 g0, g1, s0, s1, m):
    cid = lax.axis_index("c")
    sid = lax.axis_index("s")
    wid = sid * NC + cid
    r0 = sid * ROWS_PER_TILE
    ebase = wid * EPT
    pltpu.sync_copy(zeros_hbm, acc.at[pl.ds(r0, ROWS_PER_TILE)])
    plsc.subcore_barrier()

    def mul(rows_v, ex_v):
        for g in range(K // 16):
            for e in range(16):
                row = g * 16 + e
                for hh in range(HEADS):
                    w = plsc.load_gather(
                        ex_v, [jnp.full((16,), row, _i32),
                               jnp.full((16,), hh, _i32)])
                    for d2 in range(2):
                        c0 = (hh * 2 + d2) * 16
                        rows_v[row, pl.ds(c0, 16)] = rows_v[row, pl.ds(c0, 16)] * w

    def ids(i, src_v, dst_v):
        pltpu.async_copy(src_hbm.at[pl.ds(ebase + i * K, K)], src_v, m)
        pltpu.async_copy(dst_hbm.at[pl.ds(ebase + i * K, K)], dst_v, m)

    def idwait(i, src_v, dst_v):
        pltpu.make_async_copy(src_hbm.at[pl.ds(ebase + i * K, K)], src_v,
                              m).wait()
        pltpu.make_async_copy(dst_hbm.at[pl.ds(ebase + i * K, K)], dst_v,
                              m).wait()

    def gather(i, src_v, rows_v, ex_v, gsem):
        pltpu.async_copy(h_hbm.at[src_v], rows_v, gsem)
        pltpu.async_copy(ex_hbm.at[pl.ds(ebase + i * K, K)], ex_v, gsem)

    def gwait(i, src_v, rows_v, ex_v, gsem):
        pltpu.make_async_copy(h_hbm.at[src_v], rows_v, gsem).wait()
        pltpu.make_async_copy(ex_hbm.at[pl.ds(ebase + i * K, K)], ex_v,
                              gsem).wait()

    def scat(rows_v, dst_v, ssem):
        pltpu.async_copy(rows_v, acc.at[dst_v], ssem, add=True)

    def swait(rows_v, dst_v, ssem):
        pltpu.make_async_copy(rows_v, acc.at[dst_v], ssem).wait()

    pltpu.sync_copy(src_hbm.at[pl.ds(ebase, K)], src0)
    pltpu.sync_copy(dst_hbm.at[pl.ds(ebase, K)], dst0)
    gather(0, src0, rows0, ex0, g0)
    ids(1, src1, dst1)
    gwait(0, src0, rows0, ex0, g0)
    idwait(1, src1, dst1)
    gather(1, src1, rows1, ex1, g1)
    mul(rows0, ex0)
    scat(rows0, dst0, s0)

    def chunk(c, src_c, dst_c, rows_c, ex_c, gc, sc,
              src_n, dst_n, rows_n, ex_n, gn, sn, last):
        swait(rows_n, dst_n, sn)
        if last is None:
            ids(c + 1, src_n, dst_n)
        gwait(c, src_c, rows_c, ex_c, gc)
        if last is None:
            idwait(c + 1, src_n, dst_n)
            gather(c + 1, src_n, rows_n, ex_n, gn)
        mul(rows_c, ex_c)
        scat(rows_c, dst_c, sc)

    def body(j, carry):
        c1 = 2 * j + 1
        chunk(c1, src1, dst1, rows1, ex1, g1, s1,
              src0, dst0, rows0, ex0, g0, s0, None)
        c2 = 2 * j + 2
        swait(rows1, dst1, s1)

        @pl.when(j != NCHUNK // 2 - 1)
        def _():
            ids(c2 + 1, src1, dst1)

        gwait(c2, src0, rows0, ex0, g0)

        @pl.when(j != NCHUNK // 2 - 1)
        def _():
            idwait(c2 + 1, src1, dst1)
            gather(c2 + 1, src1, rows1, ex1, g1)

        mul(rows0, ex0)
        scat(rows0, dst0, s0)
        return carry

    lax.fori_loop(0, NCHUNK // 2, body, 0)
    swait(rows0, dst0, s0)
    plsc.subcore_barrier()
    pltpu.sync_copy(acc.at[pl.ds(r0, ROWS_PER_TILE)],
                    num_hbm.at[cid, pl.ds(r0, ROWS_PER_TILE)])


def _sc_sage_body(h_hbm, src_hbm, dst_hbm, zeros_hbm, s_hbm,
                  src0, src1, dst0, dst1, rows0, rows1, acc,
                  g0, g1, s0, s1, m):
    cid = lax.axis_index("c")
    sid = lax.axis_index("s")
    wid = sid * NC + cid
    r0 = sid * ROWS_PER_TILE
    ebase = wid * EPT
    pltpu.sync_copy(zeros_hbm, acc.at[pl.ds(r0, ROWS_PER_TILE)])
    plsc.subcore_barrier()

    def ids(i, src_v, dst_v):
        pltpu.async_copy(src_hbm.at[pl.ds(ebase + i * K, K)], src_v, m)
        pltpu.async_copy(dst_hbm.at[pl.ds(ebase + i * K, K)], dst_v, m)

    def idwait(i, src_v, dst_v):
        pltpu.make_async_copy(src_hbm.at[pl.ds(ebase + i * K, K)], src_v,
                              m).wait()
        pltpu.make_async_copy(dst_hbm.at[pl.ds(ebase + i * K, K)], dst_v,
                              m).wait()

    def gather(src_v, rows_v, gsem):
        pltpu.async_copy(h_hbm.at[src_v], rows_v, gsem)

    def gwait(src_v, rows_v, gsem):
        pltpu.make_async_copy(h_hbm.at[src_v], rows_v, gsem).wait()

    def scat(rows_v, dst_v, ssem):
        pltpu.async_copy(rows_v, acc.at[dst_v], ssem, add=True)

    def swait(rows_v, dst_v, ssem):
        pltpu.make_async_copy(rows_v, acc.at[dst_v], ssem).wait()

    pltpu.sync_copy(src_hbm.at[pl.ds(ebase, K)], src0)
    pltpu.sync_copy(dst_hbm.at[pl.ds(ebase, K)], dst0)
    gather(src0, rows0, g0)
    ids(1, src1, dst1)
    gwait(src0, rows0, g0)
    idwait(1, src1, dst1)
    gather(src1, rows1, g1)
    scat(rows0, dst0, s0)

    def body(j, carry):
        c1 = 2 * j + 1
        swait(rows0, dst0, s0)
        ids(c1 + 1, src0, dst0)
        gwait(src1, rows1, g1)
        idwait(c1 + 1, src0, dst0)
        gather(src0, rows0, g0)
        scat(rows1, dst1, s1)
        c2 = 2 * j + 2
        swait(rows1, dst1, s1)

        @pl.when(j != NCHUNK // 2 - 1)
        def _():
            ids(c2 + 1, src1, dst1)

        gwait(src0, rows0, g0)

        @pl.when(j != NCHUNK // 2 - 1)
        def _():
            idwait(c2 + 1, src1, dst1)
            gather(src1, rows1, g1)

        scat(rows0, dst0, s0)
        return carry

    lax.fori_loop(0, NCHUNK // 2, body, 0)
    swait(rows0, dst0, s0)
    plsc.subcore_barrier()
    pltpu.sync_copy(acc.at[pl.ds(r0, ROWS_PER_TILE)],
                    s_hbm.at[cid, pl.ds(r0, ROWS_PER_TILE)])


def _make_sc_mesh():
    return plsc.VectorSubcoreMesh(core_axis_name="c", subcore_axis_name="s")


_SC_PARAMS = pltpu.CompilerParams(needs_layout_passes=False)


def _sc_gatex(asrc_flat, adst_flat, src, dst, zeros):
    return pl.kernel(
        _sc_gatex_body,
        out_type=[
            jax.ShapeDtypeStruct((N_EDGES, 16), _f32),
            jax.ShapeDtypeStruct((NC, DEN_ROWS, D), _f32),
        ],
        mesh=_make_sc_mesh(),
        scratch_types=[
            pltpu.VMEM((K,), _i32),
            pltpu.VMEM((K,), _i32),
            pltpu.VMEM((K,), _i32),
            pltpu.VMEM((K, 16), _f32),
            pltpu.VMEM((K, D), _f32),
            pltpu.VMEM((HEADS * N_NODES,), _f32),
            pltpu.VMEM((HEADS * N_NODES,), _f32),
            pltpu.VMEM_SHARED((DEN_ROWS, D), _f32),
        ],
        compiler_params=_SC_PARAMS,
    )(asrc_flat, adst_flat, src, dst, zeros)


def _sc_gat(h, ex, src, dst, zeros):
    return pl.kernel(
        _sc_gat_body,
        out_type=jax.ShapeDtypeStruct((NC, N_PAD, D), _f32),
        mesh=_make_sc_mesh(),
        scratch_types=[
            pltpu.VMEM((K,), _i32),
            pltpu.VMEM((K,), _i32),
            pltpu.VMEM((K,), _i32),
            pltpu.VMEM((K,), _i32),
            pltpu.VMEM((K, D), _f32),
            pltpu.VMEM((K, D), _f32),
            pltpu.VMEM((K, 16), _f32),
            pltpu.VMEM((K, 16), _f32),
            pltpu.VMEM_SHARED((N_PAD, D), _f32),
            pltpu.SemaphoreType.DMA,
            pltpu.SemaphoreType.DMA,
            pltpu.SemaphoreType.DMA,
            pltpu.SemaphoreType.DMA,
            pltpu.SemaphoreType.DMA,
        ],
        compiler_params=_SC_PARAMS,
    )(h, ex, src, dst, zeros)


def _sc_sage(h, src, dst, zeros):
    return pl.kernel(
        _sc_sage_body,
        out_type=jax.ShapeDtypeStruct((NC, N_PAD, D), _f32),
        mesh=_make_sc_mesh(),
        scratch_types=[
            pltpu.VMEM((K,), _i32),
            pltpu.VMEM((K,), _i32),
            pltpu.VMEM((K,), _i32),
            pltpu.VMEM((K,), _i32),
            pltpu.VMEM((K, D), _f32),
            pltpu.VMEM((K, D), _f32),
            pltpu.VMEM_SHARED((N_PAD, D), _f32),
            pltpu.SemaphoreType.DMA,
            pltpu.SemaphoreType.DMA,
            pltpu.SemaphoreType.DMA,
            pltpu.SemaphoreType.DMA,
            pltpu.SemaphoreType.DMA,
        ],
        compiler_params=_SC_PARAMS,
    )(h, src, dst, zeros)


_HI = dict(preferred_element_type=_f32, precision=lax.Precision.HIGHEST)
BROW = 2000
GRID = N_NODES // BROW


def _full(shape):
    return pl.BlockSpec(shape, lambda i: (0,) * len(shape))


def _rows(minor):
    return pl.BlockSpec((BROW, minor), lambda i: (i, 0))


def _prows(minor):
    return pl.BlockSpec((NC, BROW, minor), lambda i: (0, i, 0))


def _tc_pre_body(x_ref, w_ref, ms_ref, md_ref, wr_ref, br_ref,
                 h_ref, as_ref, ad_ref, res_ref):
    x = x_ref[...]
    h = jnp.dot(x, w_ref[...], **_HI)
    h_ref[...] = h
    as_ref[...] = jnp.dot(h, ms_ref[...], **_HI)
    ad_ref[...] = jnp.dot(h, md_ref[...], **_HI)
    res_ref[...] = jnp.dot(x, wr_ref[...], **_HI) + br_ref[...]


def _tc_pre(x, w, ms, md, wr, br):
    return pl.pallas_call(
        _tc_pre_body,
        grid=(GRID,),
        in_specs=[_rows(D), _full((D, D)), _full((D, HEADS)),
                  _full((D, HEADS)), _full((D, D)), _full((1, D))],
        out_specs=[_rows(D), _rows(HEADS), _rows(HEADS), _rows(D)],
        out_shape=[
            jax.ShapeDtypeStruct((N_NODES, D), _f32),
            jax.ShapeDtypeStruct((N_NODES, HEADS), _f32),
            jax.ShapeDtypeStruct((N_NODES, HEADS), _f32),
            jax.ShapeDtypeStruct((N_NODES, D), _f32),
        ],
    )(x, w, ms, md, wr, br)


def _bn_lrelu(out, g, b, res):
    m = jnp.mean(out, axis=0, keepdims=True)
    v = jnp.mean((out - m) * (out - m), axis=0, keepdims=True)
    out = (out - m) / jnp.sqrt(v + 1e-5) * g + b
    out = out + res
    return jnp.where(out > 0, out, 0.2 * out)


def _tc_bnres_body(o_ref, g_ref, bb_ref, res_ref, hf_ref):
    hf_ref[...] = _bn_lrelu(o_ref[...], g_ref[...], bb_ref[...], res_ref[...])


def _tc_bnres(out, g, bb, res):
    return pl.pallas_call(
        _tc_bnres_body,
        out_shape=jax.ShapeDtypeStruct((N_NODES, D), _f32),
    )(out, g, bb, res)


def _tc_bn_body(o_ref, g_ref, bb_ref, hf_ref):
    hf_ref[...] = _bn_lrelu(o_ref[...], g_ref[...], bb_ref[...], 0.0)


def _tc_bn(out, g, bb):
    return pl.pallas_call(
        _tc_bn_body,
        out_shape=jax.ShapeDtypeStruct((N_NODES, D), _f32),
    )(out, g, bb)


def _tc_gatcomb_body(nump, denp, h_ref, as_ref, ad_ref, e_ref, eh_ref,
                     b_ref, o_ref, den_ref):
    num = nump[0] + nump[1]
    den16 = denp[0] + denp[1]
    aself = as_ref[...] + ad_ref[...]
    ex4 = jnp.exp(jnp.where(aself > 0, aself, 0.2 * aself))
    exx = jnp.dot(ex4, eh_ref[...], **_HI)
    numt = num + h_ref[...] * exx
    denx = jnp.dot(den16, e_ref[...], **_HI) + exx
    o_ref[...] = numt / (denx + 1e-16) + b_ref[...]
    den_ref[...] = den16


def _tc_gatcomb(nump, denp, h, as4, ad4, e, eh, b):
    return pl.pallas_call(
        _tc_gatcomb_body,
        grid=(GRID,),
        in_specs=[_prows(D), _prows(16), _rows(D), _rows(HEADS), _rows(HEADS),
                  _full((16, D)), _full((HEADS, D)), _full((1, D))],
        out_specs=[_rows(D), _rows(16)],
        out_shape=[
            jax.ShapeDtypeStruct((N_NODES, D), _f32),
            jax.ShapeDtypeStruct((N_NODES, 16), _f32),
        ],
    )(nump, denp, h, as4, ad4, e, eh, b)


def _tc_sagecomb_body(sp, den_ref, e4_ref, hin_ref, wl_ref, bl_ref, wr_ref,
                      o_ref):
    s = sp[0] + sp[1]
    cntx = jnp.dot(den_ref[...], e4_ref[...], **_HI)
    mean = s / jnp.maximum(cntx, 1.0)
    o_ref[...] = (jnp.dot(mean, wl_ref[...], **_HI) + bl_ref[...]
                  + jnp.dot(hin_ref[...], wr_ref[...], **_HI))


def _tc_sagecomb(sp, den16, e4, hin, wl, bl, wr):
    return pl.pallas_call(
        _tc_sagecomb_body,
        grid=(GRID,),
        in_specs=[_prows(D), _rows(16), _full((16, D)), _rows(D),
                  _full((D, D)), _full((1, D)), _full((D, D))],
        out_specs=_rows(D),
        out_shape=jax.ShapeDtypeStruct((N_NODES, D), _f32),
    )(sp, den16, e4, hin, wl, bl, wr)


def _tc_pool_body(h_ref, batch_ref, sums_ref, cnt_ref):
    i = pl.program_id(0)
    bt = batch_ref[...]
    oh = (bt == lax.broadcasted_iota(_i32, (1, N_GRAPHS), 1)).astype(_f32)
    part = lax.dot_general(oh, h_ref[...], (((0,), (0,)), ((), ())), **_HI)
    ones = jnp.ones((BROW, N_GRAPHS), _f32)
    pcnt = lax.dot_general(oh, ones, (((0,), (0,)), ((), ())),
                           preferred_element_type=_f32)

    @pl.when(i == 0)
    def _():
        sums_ref[...] = jnp.zeros((N_GRAPHS, D), _f32)
        cnt_ref[...] = jnp.zeros((N_GRAPHS, N_GRAPHS), _f32)

    sums_ref[...] += part
    cnt_ref[...] += pcnt


def _tc_pool(h5, batch2d):
    return pl.pallas_call(
        _tc_pool_body,
        grid=(GRID,),
        in_specs=[_rows(D), _rows(1)],
        out_specs=[pl.BlockSpec((N_GRAPHS, D), lambda i: (0, 0)),
                   pl.BlockSpec((N_GRAPHS, N_GRAPHS), lambda i: (0, 0))],
        out_shape=[
            jax.ShapeDtypeStruct((N_GRAPHS, D), _f32),
            jax.ShapeDtypeStruct((N_GRAPHS, N_GRAPHS), _f32),
        ],
    )(h5, batch2d)


def _tc_fc_body(sums_ref, cnt_ref, w_ref, b_ref, out_ref):
    cnt = cnt_ref[:, :1]
    gm = sums_ref[...] / jnp.maximum(cnt, 1.0)
    out_ref[...] = jnp.dot(gm, w_ref[...], **_HI) + b_ref[...]


def _tc_fc(sums, cnt, w, b):
    return pl.pallas_call(
        _tc_fc_body,
        out_shape=jax.ShapeDtypeStruct((N_GRAPHS, w.shape[1]), _f32),
    )(sums, cnt, w, b)


def _att_mat(att):
    a = att.reshape(HEADS, CH)
    eye = jnp.eye(HEADS, dtype=_f32)
    return jnp.einsum('hc,hk->hck', a, eye).reshape(D, HEADS)


_E16 = np.zeros((16, D), np.float32)
for _h in range(HEADS):
    _E16[_h, _h * CH:(_h + 1) * CH] = 1.0

_EH = np.zeros((HEADS, D), np.float32)
for _h in range(HEADS):
    _EH[_h, _h * CH:(_h + 1) * CH] = 1.0

_E4 = np.zeros((16, D), np.float32)
_E4[HEADS, :] = 1.0


def kernel(x, edge_index, batch, params):
    p = params
    src = edge_index[0]
    dst = edge_index[1]
    zeros = jnp.zeros((ROWS_PER_TILE, D), _f32)
    e16 = jnp.asarray(_E16)
    eh = jnp.asarray(_EH)
    e4 = jnp.asarray(_E4)

    def row(v):
        return v.reshape(1, -1)

    h1, as1, ad1, res1 = _tc_pre(x, p['gat1']['W'],
                                 _att_mat(p['gat1']['att_src']),
                                 _att_mat(p['gat1']['att_dst']),
                                 p['res1']['W'], row(p['res1']['b']))
    ex1, den1 = _sc_gatex(as1.reshape(-1), ad1.reshape(-1), src, dst, zeros)
    num1 = _sc_gat(h1, ex1, src, dst, zeros)
    den1 = den1.reshape(NC, N_PAD, 16)
    o1, den16 = _tc_gatcomb(num1, den1, h1, as1, ad1, e16, eh,
                            row(p['gat1']['b']))
    h1f = _tc_bnres(o1, row(p['bn1']['g']), row(p['bn1']['b']), res1)

    h2, as2, ad2, res2 = _tc_pre(h1f, p['gat2']['W'],
                                 _att_mat(p['gat2']['att_src']),
                                 _att_mat(p['gat2']['att_dst']),
                                 p['res2']['W'], row(p['res2']['b']))
    ex2, den2 = _sc_gatex(as2.reshape(-1), ad2.reshape(-1), src, dst, zeros)
    num2 = _sc_gat(h2, ex2, src, dst, zeros)
    den2 = den2.reshape(NC, N_PAD, 16)
    o2, _ = _tc_gatcomb(num2, den2, h2, as2, ad2, e16, eh,
                        row(p['gat2']['b']))
    h2f = _tc_bnres(o2, row(p['bn2']['g']), row(p['bn2']['b']), res2)

    h = h2f
    for name, bn in (('sage3', 'bn3'), ('sage4', 'bn4'), ('sage5', 'bn5')):
        sp = _sc_sage(h, src, dst, zeros)
        o = _tc_sagecomb(sp, den16, e4, h,
                         p[name]['Wl'], row(p[name]['bl']), p[name]['Wr'])
        h = _tc_bn(o, row(p[bn]['g']), row(p[bn]['b']))

    sums, cnt = _tc_pool(h, batch.reshape(-1, 1))
    return _tc_fc(sums, cnt, p['fc']['W'], row(p['fc']['b']))


# speedup vs baseline: 45.7348x; 1.3638x over previous
import functools

import jax
import jax.numpy as jnp
import numpy as np
from jax import lax
from jax.experimental import pallas as pl
from jax.experimental.pallas import tpu as pltpu
from jax.experimental.pallas import tpu_sc as plsc

N_NODES = 10000
N_PAD = 10240
N_EDGES = 320000
D = 128
HEADS = 4
CH = 32
N_GRAPHS = 16

NC = 2
NS = 16
NW = NC * NS
K = 80
EPT = N_EDGES // NW
NCHUNK = EPT // K
ROWS_PER_TILE = N_PAD // NS

_f32 = jnp.float32
_i32 = jnp.int32


DEN_ROWS = N_PAD // 8
DEN_TILE = DEN_ROWS // NS


def _sc_gatex_body(asrc_hbm, adst_hbm, src_hbm, dst_hbm, zeros_hbm,
                   ex_hbm, den_hbm,
                   src0, src1, dst0, dst1, drow_v, ex0, ex1, exw_v,
                   asrc_t, adst_t, accden, m, x0, x1):
    cid = lax.axis_index("c")
    sid = lax.axis_index("s")
    wid = sid * NC + cid
    ebase = wid * EPT
    pltpu.sync_copy(asrc_hbm, asrc_t)
    pltpu.sync_copy(adst_hbm, adst_t)
    pltpu.sync_copy(zeros_hbm.at[pl.ds(0, DEN_TILE)],
                    accden.at[pl.ds(sid * DEN_TILE, DEN_TILE)])
    pltpu.sync_copy(zeros_hbm.at[pl.ds(0, K)], exw_v)
    plsc.subcore_barrier()

    lanes = lax.iota(_i32, 16)
    ones16 = jnp.ones((16,), _f32)
    zeros16v = jnp.zeros((16,), _f32)

    def ids(i, src_v, dst_v):
        pltpu.async_copy(src_hbm.at[pl.ds(ebase + i * K, K)], src_v, m)
        pltpu.async_copy(dst_hbm.at[pl.ds(ebase + i * K, K)], dst_v, m)

    def idwait(i, src_v, dst_v):
        pltpu.make_async_copy(src_hbm.at[pl.ds(ebase + i * K, K)], src_v,
                              m).wait()
        pltpu.make_async_copy(dst_hbm.at[pl.ds(ebase + i * K, K)], dst_v,
                              m).wait()

    def compute(i, src_v, dst_v, ex_v):
        for g in range(K // 16):
            ridx = lanes + (g * 16)
            sids = src_v[pl.ds(g * 16, 16)] * 4
            dvals = dst_v[pl.ds(g * 16, 16)]
            dids = dvals * 4
            drow_v[pl.ds(g * 16, 16)] = lax.shift_right_logical(dvals, 3)
            cbase = (dvals & 7) * 16
            for hh in range(HEADS):
                s_ = plsc.load_gather(asrc_t, [sids + hh])
                d_ = plsc.load_gather(adst_t, [dids + hh])
                al = s_ + d_
                al = jnp.where(al > 0, al, 0.2 * al)
                exv = jnp.exp(al)
                plsc.store_scatter(ex_v, [ridx, jnp.full((16,), hh, _i32)], exv)
                plsc.store_scatter(exw_v, [ridx, cbase + hh], exv)
            plsc.store_scatter(ex_v, [ridx, jnp.full((16,), HEADS, _i32)], ones16)
            plsc.store_scatter(exw_v, [ridx, cbase + HEADS], ones16)
        pltpu.sync_copy(exw_v, accden.at[drow_v], add=True)
        for g in range(K // 16):
            ridx = lanes + (g * 16)
            cbase = (dst_v[pl.ds(g * 16, 16)] & 7) * 16
            for hh in range(HEADS + 1):
                plsc.store_scatter(exw_v, [ridx, cbase + hh], zeros16v)

    def exwrite(i, ex_v, xsem):
        pltpu.async_copy(ex_v, ex_hbm.at[pl.ds(ebase + i * K, K)], xsem)

    def exdrain(i, ex_v, xsem):
        pltpu.make_async_copy(ex_v, ex_hbm.at[pl.ds(ebase + i * K, K)],
                              xsem).wait()

    pltpu.sync_copy(src_hbm.at[pl.ds(ebase, K)], src0)
    pltpu.sync_copy(dst_hbm.at[pl.ds(ebase, K)], dst0)
    ids(1, src1, dst1)
    compute(0, src0, dst0, ex0)
    exwrite(0, ex0, x0)

    def body(j, carry):
        c1 = 2 * j + 1
        ids(c1 + 1, src0, dst0)

        @pl.when(j > 0)
        def _():
            exdrain(c1 - 2, ex1, x1)

        idwait(c1, src1, dst1)
        compute(c1, src1, dst1, ex1)
        exwrite(c1, ex1, x1)

        c2 = 2 * j + 2

        @pl.when(j != NCHUNK // 2 - 1)
        def _():
            ids(c2 + 1, src1, dst1)

        exdrain(c2 - 2, ex0, x0)
        idwait(c2, src0, dst0)
        compute(c2, src0, dst0, ex0)
        exwrite(c2, ex0, x0)
        return carry

    lax.fori_loop(0, NCHUNK // 2, body, 0)
    exdrain(NCHUNK - 1, ex0, x0)
    exdrain(NCHUNK - 2, ex1, x1)
    plsc.subcore_barrier()
    pltpu.sync_copy(accden.at[pl.ds(sid * DEN_TILE, DEN_TILE)],
                    den_hbm.at[cid, pl.ds(sid * DEN_TILE, DEN_TILE)])


def _sc_gat_body(h_hbm, ex_hbm, src_hbm, dst_hbm, zeros_hbm, num_hbm,
                 src0, src1, dst0, dst1, rows0, rows1, ex0, ex1, acc,
                 g0, g1, s0, s1, m):
    cid = lax.axis_index("c")
    sid = lax.axis_index("s")
    wid = sid * NC + cid
    r0 = sid * ROWS_PER_TILE
    ebase = wid * EPT
    pltpu.sync_copy(zeros_hbm, acc.at[pl.ds(r0, ROWS_PER_TILE)])
    plsc.subcore_barrier()

    def mul(rows_v, ex_v):
        for g in range(K // 16):
            for e in range(16):
                row = g * 16 + e
                ws = [plsc.load_gather(
                    ex_v, [jnp.full((16,), row, _i32),
                           jnp.full((16,), hh, _i32)])
                      for hh in range(HEADS)]
                for hh in range(HEADS):
                    for d2 in range(2):
                        c0 = (hh * 2 + d2) * 16
                        rows_v[row, pl.ds(c0, 16)] = (
                            rows_v[row, pl.ds(c0, 16)] * ws[hh])

    def ids(i, src_v, dst_v):
        pltpu.async_copy(src_hbm.at[pl.ds(ebase + i * K, K)], src_v, m)
        pltpu.async_copy(dst_hbm.at[pl.ds(ebase + i * K, K)], dst_v, m)

    def idwait(i, src_v, dst_v):
        pltpu.make_async_copy(src_hbm.at[pl.ds(ebase + i * K, K)], src_v,
                              m).wait()
        pltpu.make_async_copy(dst_hbm.at[pl.ds(ebase + i * K, K)], dst_v,
                              m).wait()

    def gather(i, src_v, rows_v, ex_v, gsem):
        pltpu.async_copy(h_hbm.at[src_v], rows_v, gsem)
        pltpu.async_copy(ex_hbm.at[pl.ds(ebase + i * K, K)], ex_v, gsem)

    def gwait(i, src_v, rows_v, ex_v, gsem):
        pltpu.make_async_copy(h_hbm.at[src_v], rows_v, gsem).wait()
        pltpu.make_async_copy(ex_hbm.at[pl.ds(ebase + i * K, K)], ex_v,
                              gsem).wait()

    def scat(rows_v, dst_v, ssem):
        pltpu.async_copy(rows_v, acc.at[dst_v], ssem, add=True)

    def swait(rows_v, dst_v, ssem):
        pltpu.make_async_copy(rows_v, acc.at[dst_v], ssem).wait()

    pltpu.sync_copy(src_hbm.at[pl.ds(ebase, K)], src0)
    pltpu.sync_copy(dst_hbm.at[pl.ds(ebase, K)], dst0)
    gather(0, src0, rows0, ex0, g0)
    ids(1, src1, dst1)
    gwait(0, src0, rows0, ex0, g0)
    idwait(1, src1, dst1)
    gather(1, src1, rows1, ex1, g1)
    mul(rows0, ex0)
    scat(rows0, dst0, s0)

    def chunk(c, src_c, dst_c, rows_c, ex_c, gc, sc,
              src_n, dst_n, rows_n, ex_n, gn, sn, last):
        swait(rows_n, dst_n, sn)
        if last is None:
            ids(c + 1, src_n, dst_n)
        gwait(c, src_c, rows_c, ex_c, gc)
        if last is None:
            idwait(c + 1, src_n, dst_n)
            gather(c + 1, src_n, rows_n, ex_n, gn)
        mul(rows_c, ex_c)
        scat(rows_c, dst_c, sc)

    def body(j, carry):
        c1 = 2 * j + 1
        chunk(c1, src1, dst1, rows1, ex1, g1, s1,
              src0, dst0, rows0, ex0, g0, s0, None)
        c2 = 2 * j + 2
        swait(rows1, dst1, s1)

        @pl.when(j != NCHUNK // 2 - 1)
        def _():
            ids(c2 + 1, src1, dst1)

        gwait(c2, src0, rows0, ex0, g0)

        @pl.when(j != NCHUNK // 2 - 1)
        def _():
            idwait(c2 + 1, src1, dst1)
            gather(c2 + 1, src1, rows1, ex1, g1)

        mul(rows0, ex0)
        scat(rows0, dst0, s0)
        return carry

    lax.fori_loop(0, NCHUNK // 2, body, 0)
    swait(rows0, dst0, s0)
    plsc.subcore_barrier()
    pltpu.sync_copy(acc.at[pl.ds(r0, ROWS_PER_TILE)],
                    num_hbm.at[cid, pl.ds(r0, ROWS_PER_TILE)])


def _sc_sage_body(h_hbm, src_hbm, dst_hbm, zeros_hbm, s_hbm,
                  src0, src1, dst0, dst1, rows0, rows1, acc,
                  g0, g1, s0, s1, m):
    cid = lax.axis_index("c")
    sid = lax.axis_index("s")
    wid = sid * NC + cid
    r0 = sid * ROWS_PER_TILE
    ebase = wid * EPT
    pltpu.sync_copy(zeros_hbm, acc.at[pl.ds(r0, ROWS_PER_TILE)])
    plsc.subcore_barrier()

    def ids(i, src_v, dst_v):
        pltpu.async_copy(src_hbm.at[pl.ds(ebase + i * K, K)], src_v, m)
        pltpu.async_copy(dst_hbm.at[pl.ds(ebase + i * K, K)], dst_v, m)

    def idwait(i, src_v, dst_v):
        pltpu.make_async_copy(src_hbm.at[pl.ds(ebase + i * K, K)], src_v,
                              m).wait()
        pltpu.make_async_copy(dst_hbm.at[pl.ds(ebase + i * K, K)], dst_v,
                              m).wait()

    def gather(src_v, rows_v, gsem):
        pltpu.async_copy(h_hbm.at[src_v], rows_v, gsem)

    def gwait(src_v, rows_v, gsem):
        pltpu.make_async_copy(h_hbm.at[src_v], rows_v, gsem).wait()

    def scat(rows_v, dst_v, ssem):
        pltpu.async_copy(rows_v, acc.at[dst_v], ssem, add=True)

    def swait(rows_v, dst_v, ssem):
        pltpu.make_async_copy(rows_v, acc.at[dst_v], ssem).wait()

    pltpu.sync_copy(src_hbm.at[pl.ds(ebase, K)], src0)
    pltpu.sync_copy(dst_hbm.at[pl.ds(ebase, K)], dst0)
    gather(src0, rows0, g0)
    ids(1, src1, dst1)
    gwait(src0, rows0, g0)
    idwait(1, src1, dst1)
    gather(src1, rows1, g1)
    scat(rows0, dst0, s0)

    def body(j, carry):
        c1 = 2 * j + 1
        swait(rows0, dst0, s0)
        ids(c1 + 1, src0, dst0)
        gwait(src1, rows1, g1)
        idwait(c1 + 1, src0, dst0)
        gather(src0, rows0, g0)
        scat(rows1, dst1, s1)
        c2 = 2 * j + 2
        swait(rows1, dst1, s1)

        @pl.when(j != NCHUNK // 2 - 1)
        def _():
            ids(c2 + 1, src1, dst1)

        gwait(src0, rows0, g0)

        @pl.when(j != NCHUNK // 2 - 1)
        def _():
            idwait(c2 + 1, src1, dst1)
            gather(src1, rows1, g1)

        scat(rows0, dst0, s0)
        return carry

    lax.fori_loop(0, NCHUNK // 2, body, 0)
    swait(rows0, dst0, s0)
    plsc.subcore_barrier()
    pltpu.sync_copy(acc.at[pl.ds(r0, ROWS_PER_TILE)],
                    s_hbm.at[cid, pl.ds(r0, ROWS_PER_TILE)])


def _make_sc_mesh():
    return plsc.VectorSubcoreMesh(core_axis_name="c", subcore_axis_name="s")


_SC_PARAMS = pltpu.CompilerParams(needs_layout_passes=False)


def _sc_gatex(asrc_flat, adst_flat, src, dst, zeros):
    return pl.kernel(
        _sc_gatex_body,
        out_type=[
            jax.ShapeDtypeStruct((N_EDGES, 16), _f32),
            jax.ShapeDtypeStruct((NC, DEN_ROWS, D), _f32),
        ],
        mesh=_make_sc_mesh(),
        scratch_types=[
            pltpu.VMEM((K,), _i32),
            pltpu.VMEM((K,), _i32),
            pltpu.VMEM((K,), _i32),
            pltpu.VMEM((K,), _i32),
            pltpu.VMEM((K,), _i32),
            pltpu.VMEM((K, 16), _f32),
            pltpu.VMEM((K, 16), _f32),
            pltpu.VMEM((K, D), _f32),
            pltpu.VMEM((HEADS * N_NODES,), _f32),
            pltpu.VMEM((HEADS * N_NODES,), _f32),
            pltpu.VMEM_SHARED((DEN_ROWS, D), _f32),
            pltpu.SemaphoreType.DMA,
            pltpu.SemaphoreType.DMA,
            pltpu.SemaphoreType.DMA,
        ],
        compiler_params=_SC_PARAMS,
    )(asrc_flat, adst_flat, src, dst, zeros)


def _sc_gat(h, ex, src, dst, zeros):
    return pl.kernel(
        _sc_gat_body,
        out_type=jax.ShapeDtypeStruct((NC, N_PAD, D), _f32),
        mesh=_make_sc_mesh(),
        scratch_types=[
            pltpu.VMEM((K,), _i32),
            pltpu.VMEM((K,), _i32),
            pltpu.VMEM((K,), _i32),
            pltpu.VMEM((K,), _i32),
            pltpu.VMEM((K, D), _f32),
            pltpu.VMEM((K, D), _f32),
            pltpu.VMEM((K, 16), _f32),
            pltpu.VMEM((K, 16), _f32),
            pltpu.VMEM_SHARED((N_PAD, D), _f32),
            pltpu.SemaphoreType.DMA,
            pltpu.SemaphoreType.DMA,
            pltpu.SemaphoreType.DMA,
            pltpu.SemaphoreType.DMA,
            pltpu.SemaphoreType.DMA,
        ],
        compiler_params=_SC_PARAMS,
    )(h, ex, src, dst, zeros)


def _sc_sage(h, src, dst, zeros):
    return pl.kernel(
        _sc_sage_body,
        out_type=jax.ShapeDtypeStruct((NC, N_PAD, D), _f32),
        mesh=_make_sc_mesh(),
        scratch_types=[
            pltpu.VMEM((K,), _i32),
            pltpu.VMEM((K,), _i32),
            pltpu.VMEM((K,), _i32),
            pltpu.VMEM((K,), _i32),
            pltpu.VMEM((K, D), _f32),
            pltpu.VMEM((K, D), _f32),
            pltpu.VMEM_SHARED((N_PAD, D), _f32),
            pltpu.SemaphoreType.DMA,
            pltpu.SemaphoreType.DMA,
            pltpu.SemaphoreType.DMA,
            pltpu.SemaphoreType.DMA,
            pltpu.SemaphoreType.DMA,
        ],
        compiler_params=_SC_PARAMS,
    )(h, src, dst, zeros)


_HI = dict(preferred_element_type=_f32, precision=lax.Precision.HIGHEST)
BROW = 2000
GRID = N_NODES // BROW


def _full(shape):
    return pl.BlockSpec(shape, lambda i: (0,) * len(shape))


def _rows(minor):
    return pl.BlockSpec((BROW, minor), lambda i: (i, 0))


def _prows(minor):
    return pl.BlockSpec((NC, BROW, minor), lambda i: (0, i, 0))


def _tc_pre_body(x_ref, w_ref, ms_ref, md_ref, wr_ref, br_ref,
                 h_ref, as_ref, ad_ref, res_ref):
    x = x_ref[...]
    h = jnp.dot(x, w_ref[...], **_HI)
    h_ref[...] = h
    as_ref[...] = jnp.dot(h, ms_ref[...], **_HI)
    ad_ref[...] = jnp.dot(h, md_ref[...], **_HI)
    res_ref[...] = jnp.dot(x, wr_ref[...], **_HI) + br_ref[...]


def _tc_pre(x, w, ms, md, wr, br):
    return pl.pallas_call(
        _tc_pre_body,
        grid=(GRID,),
        in_specs=[_rows(D), _full((D, D)), _full((D, HEADS)),
                  _full((D, HEADS)), _full((D, D)), _full((1, D))],
        out_specs=[_rows(D), _rows(HEADS), _rows(HEADS), _rows(D)],
        out_shape=[
            jax.ShapeDtypeStruct((N_NODES, D), _f32),
            jax.ShapeDtypeStruct((N_NODES, HEADS), _f32),
            jax.ShapeDtypeStruct((N_NODES, HEADS), _f32),
            jax.ShapeDtypeStruct((N_NODES, D), _f32),
        ],
    )(x, w, ms, md, wr, br)


def _bn_lrelu(out, g, b, res):
    m = jnp.mean(out, axis=0, keepdims=True)
    v = jnp.mean((out - m) * (out - m), axis=0, keepdims=True)
    out = (out - m) / jnp.sqrt(v + 1e-5) * g + b
    out = out + res
    return jnp.where(out > 0, out, 0.2 * out)


def _tc_bnres_body(o_ref, g_ref, bb_ref, res_ref, hf_ref):
    hf_ref[...] = _bn_lrelu(o_ref[...], g_ref[...], bb_ref[...], res_ref[...])


def _tc_bnres(out, g, bb, res):
    return pl.pallas_call(
        _tc_bnres_body,
        out_shape=jax.ShapeDtypeStruct((N_NODES, D), _f32),
    )(out, g, bb, res)


def _tc_bn_body(o_ref, g_ref, bb_ref, hf_ref):
    hf_ref[...] = _bn_lrelu(o_ref[...], g_ref[...], bb_ref[...], 0.0)


def _tc_bn(out, g, bb):
    return pl.pallas_call(
        _tc_bn_body,
        out_shape=jax.ShapeDtypeStruct((N_NODES, D), _f32),
    )(out, g, bb)


def _tc_gatcomb_body(nump, denp, h_ref, as_ref, ad_ref, e_ref, eh_ref,
                     b_ref, o_ref, den_ref):
    num = nump[0] + nump[1]
    den16 = denp[0] + denp[1]
    aself = as_ref[...] + ad_ref[...]
    ex4 = jnp.exp(jnp.where(aself > 0, aself, 0.2 * aself))
    exx = jnp.dot(ex4, eh_ref[...], **_HI)
    numt = num + h_ref[...] * exx
    denx = jnp.dot(den16, e_ref[...], **_HI) + exx
    o_ref[...] = numt / (denx + 1e-16) + b_ref[...]
    den_ref[...] = den16


def _tc_gatcomb(nump, denp, h, as4, ad4, e, eh, b):
    return pl.pallas_call(
        _tc_gatcomb_body,
        grid=(GRID,),
        in_specs=[_prows(D), _prows(16), _rows(D), _rows(HEADS), _rows(HEADS),
                  _full((16, D)), _full((HEADS, D)), _full((1, D))],
        out_specs=[_rows(D), _rows(16)],
        out_shape=[
            jax.ShapeDtypeStruct((N_NODES, D), _f32),
            jax.ShapeDtypeStruct((N_NODES, 16), _f32),
        ],
    )(nump, denp, h, as4, ad4, e, eh, b)


def _tc_sagecomb_body(sp, den_ref, e4_ref, hin_ref, wl_ref, bl_ref, wr_ref,
                      o_ref):
    s = sp[0] + sp[1]
    cntx = jnp.dot(den_ref[...], e4_ref[...], **_HI)
    mean = s / jnp.maximum(cntx, 1.0)
    o_ref[...] = (jnp.dot(mean, wl_ref[...], **_HI) + bl_ref[...]
                  + jnp.dot(hin_ref[...], wr_ref[...], **_HI))


def _tc_sagecomb(sp, den16, e4, hin, wl, bl, wr):
    return pl.pallas_call(
        _tc_sagecomb_body,
        grid=(GRID,),
        in_specs=[_prows(D), _rows(16), _full((16, D)), _rows(D),
                  _full((D, D)), _full((1, D)), _full((D, D))],
        out_specs=_rows(D),
        out_shape=jax.ShapeDtypeStruct((N_NODES, D), _f32),
    )(sp, den16, e4, hin, wl, bl, wr)


def _tc_pool_body(h_ref, batch_ref, sums_ref, cnt_ref):
    i = pl.program_id(0)
    bt = batch_ref[...]
    oh = (bt == lax.broadcasted_iota(_i32, (1, N_GRAPHS), 1)).astype(_f32)
    part = lax.dot_general(oh, h_ref[...], (((0,), (0,)), ((), ())), **_HI)
    ones = jnp.ones((BROW, N_GRAPHS), _f32)
    pcnt = lax.dot_general(oh, ones, (((0,), (0,)), ((), ())),
                           preferred_element_type=_f32)

    @pl.when(i == 0)
    def _():
        sums_ref[...] = jnp.zeros((N_GRAPHS, D), _f32)
        cnt_ref[...] = jnp.zeros((N_GRAPHS, N_GRAPHS), _f32)

    sums_ref[...] += part
    cnt_ref[...] += pcnt


def _tc_pool(h5, batch2d):
    return pl.pallas_call(
        _tc_pool_body,
        grid=(GRID,),
        in_specs=[_rows(D), _rows(1)],
        out_specs=[pl.BlockSpec((N_GRAPHS, D), lambda i: (0, 0)),
                   pl.BlockSpec((N_GRAPHS, N_GRAPHS), lambda i: (0, 0))],
        out_shape=[
            jax.ShapeDtypeStruct((N_GRAPHS, D), _f32),
            jax.ShapeDtypeStruct((N_GRAPHS, N_GRAPHS), _f32),
        ],
    )(h5, batch2d)


def _tc_fc_body(sums_ref, cnt_ref, w_ref, b_ref, out_ref):
    cnt = cnt_ref[:, :1]
    gm = sums_ref[...] / jnp.maximum(cnt, 1.0)
    out_ref[...] = jnp.dot(gm, w_ref[...], **_HI) + b_ref[...]


def _tc_fc(sums, cnt, w, b):
    return pl.pallas_call(
        _tc_fc_body,
        out_shape=jax.ShapeDtypeStruct((N_GRAPHS, w.shape[1]), _f32),
    )(sums, cnt, w, b)


def _att_mat(att):
    a = att.reshape(HEADS, CH)
    eye = jnp.eye(HEADS, dtype=_f32)
    return jnp.einsum('hc,hk->hck', a, eye).reshape(D, HEADS)


_E16 = np.zeros((16, D), np.float32)
for _h in range(HEADS):
    _E16[_h, _h * CH:(_h + 1) * CH] = 1.0

_EH = np.zeros((HEADS, D), np.float32)
for _h in range(HEADS):
    _EH[_h, _h * CH:(_h + 1) * CH] = 1.0

_E4 = np.zeros((16, D), np.float32)
_E4[HEADS, :] = 1.0


def kernel(x, edge_index, batch, params):
    p = params
    src = edge_index[0]
    dst = edge_index[1]
    zeros = jnp.zeros((ROWS_PER_TILE, D), _f32)
    e16 = jnp.asarray(_E16)
    eh = jnp.asarray(_EH)
    e4 = jnp.asarray(_E4)

    def row(v):
        return v.reshape(1, -1)

    h1, as1, ad1, res1 = _tc_pre(x, p['gat1']['W'],
                                 _att_mat(p['gat1']['att_src']),
                                 _att_mat(p['gat1']['att_dst']),
                                 p['res1']['W'], row(p['res1']['b']))
    ex1, den1 = _sc_gatex(as1.reshape(-1), ad1.reshape(-1), src, dst, zeros)
    num1 = _sc_gat(h1, ex1, src, dst, zeros)
    den1 = den1.reshape(NC, N_PAD, 16)
    o1, den16 = _tc_gatcomb(num1, den1, h1, as1, ad1, e16, eh,
                            row(p['gat1']['b']))
    h1f = _tc_bnres(o1, row(p['bn1']['g']), row(p['bn1']['b']), res1)

    h2, as2, ad2, res2 = _tc_pre(h1f, p['gat2']['W'],
                                 _att_mat(p['gat2']['att_src']),
                                 _att_mat(p['gat2']['att_dst']),
                                 p['res2']['W'], row(p['res2']['b']))
    ex2, den2 = _sc_gatex(as2.reshape(-1), ad2.reshape(-1), src, dst, zeros)
    num2 = _sc_gat(h2, ex2, src, dst, zeros)
    den2 = den2.reshape(NC, N_PAD, 16)
    o2, _ = _tc_gatcomb(num2, den2, h2, as2, ad2, e16, eh,
                        row(p['gat2']['b']))
    h2f = _tc_bnres(o2, row(p['bn2']['g']), row(p['bn2']['b']), res2)

    h = h2f
    for name, bn in (('sage3', 'bn3'), ('sage4', 'bn4'), ('sage5', 'bn5')):
        sp = _sc_sage(h, src, dst, zeros)
        o = _tc_sagecomb(sp, den16, e4, h,
                         p[name]['Wl'], row(p[name]['bl']), p[name]['Wr'])
        h = _tc_bn(o, row(p[bn]['g']), row(p[bn]['b']))

    sums, cnt = _tc_pool(h, batch.reshape(-1, 1))
    return _tc_fc(sums, cnt, p['fc']['W'], row(p['fc']['b']))


# speedup vs baseline: 54.5110x; 1.1919x over previous
import functools

import jax
import jax.numpy as jnp
import numpy as np
from jax import lax
from jax.experimental import pallas as pl
from jax.experimental.pallas import tpu as pltpu
from jax.experimental.pallas import tpu_sc as plsc

N_NODES = 10000
N_PAD = 10240
N_EDGES = 320000
D = 128
HEADS = 4
CH = 32
N_GRAPHS = 16

NC = 2
NS = 16
NW = NC * NS
K = 80
EPT = N_EDGES // NW
NCHUNK = EPT // K
ROWS_PER_TILE = N_PAD // NS

_f32 = jnp.float32
_i32 = jnp.int32


DEN_ROWS = N_PAD // 8
DEN_TILE = DEN_ROWS // NS


def _sc_gatex_body(asrc_hbm, adst_hbm, src_hbm, dst_hbm, zeros_hbm,
                   ex_hbm, den_hbm,
                   src0, src1, dst0, dst1, drow_v, ex0, ex1, exw_v,
                   asrc_t, adst_t, accden, m, x0, x1):
    cid = lax.axis_index("c")
    sid = lax.axis_index("s")
    wid = sid * NC + cid
    ebase = wid * EPT
    pltpu.sync_copy(asrc_hbm, asrc_t)
    pltpu.sync_copy(adst_hbm, adst_t)
    pltpu.sync_copy(zeros_hbm.at[pl.ds(0, DEN_TILE)],
                    accden.at[pl.ds(sid * DEN_TILE, DEN_TILE)])
    pltpu.sync_copy(zeros_hbm.at[pl.ds(0, K)], exw_v)
    plsc.subcore_barrier()

    lanes = lax.iota(_i32, 16)
    ones16 = jnp.ones((16,), _f32)
    zeros16v = jnp.zeros((16,), _f32)

    def ids(i, src_v, dst_v):
        pltpu.async_copy(src_hbm.at[pl.ds(ebase + i * K, K)], src_v, m)
        pltpu.async_copy(dst_hbm.at[pl.ds(ebase + i * K, K)], dst_v, m)

    def idwait(i, src_v, dst_v):
        pltpu.make_async_copy(src_hbm.at[pl.ds(ebase + i * K, K)], src_v,
                              m).wait()
        pltpu.make_async_copy(dst_hbm.at[pl.ds(ebase + i * K, K)], dst_v,
                              m).wait()

    def compute(i, src_v, dst_v, ex_v):
        for g in range(K // 16):
            ridx = lanes + (g * 16)
            sids = src_v[pl.ds(g * 16, 16)] * 4
            dvals = dst_v[pl.ds(g * 16, 16)]
            dids = dvals * 4
            drow_v[pl.ds(g * 16, 16)] = lax.shift_right_logical(dvals, 3)
            cbase = (dvals & 7) * 16
            for hh in range(HEADS):
                s_ = plsc.load_gather(asrc_t, [sids + hh])
                d_ = plsc.load_gather(adst_t, [dids + hh])
                al = s_ + d_
                al = jnp.where(al > 0, al, 0.2 * al)
                exv = jnp.exp(al)
                plsc.store_scatter(ex_v, [ridx, jnp.full((16,), hh, _i32)], exv)
                plsc.store_scatter(exw_v, [ridx, cbase + hh], exv)
            plsc.store_scatter(ex_v, [ridx, jnp.full((16,), HEADS, _i32)], ones16)
            plsc.store_scatter(exw_v, [ridx, cbase + HEADS], ones16)
        pltpu.sync_copy(exw_v, accden.at[drow_v], add=True)
        for g in range(K // 16):
            ridx = lanes + (g * 16)
            cbase = (dst_v[pl.ds(g * 16, 16)] & 7) * 16
            for hh in range(HEADS + 1):
                plsc.store_scatter(exw_v, [ridx, cbase + hh], zeros16v)

    def exwrite(i, ex_v, xsem):
        pltpu.async_copy(ex_v, ex_hbm.at[pl.ds(ebase + i * K, K)], xsem)

    def exdrain(i, ex_v, xsem):
        pltpu.make_async_copy(ex_v, ex_hbm.at[pl.ds(ebase + i * K, K)],
                              xsem).wait()

    pltpu.sync_copy(src_hbm.at[pl.ds(ebase, K)], src0)
    pltpu.sync_copy(dst_hbm.at[pl.ds(ebase, K)], dst0)
    ids(1, src1, dst1)
    compute(0, src0, dst0, ex0)
    exwrite(0, ex0, x0)

    def body(j, carry):
        c1 = 2 * j + 1
        ids(c1 + 1, src0, dst0)

        @pl.when(j > 0)
        def _():
            exdrain(c1 - 2, ex1, x1)

        idwait(c1, src1, dst1)
        compute(c1, src1, dst1, ex1)
        exwrite(c1, ex1, x1)

        c2 = 2 * j + 2

        @pl.when(j != NCHUNK // 2 - 1)
        def _():
            ids(c2 + 1, src1, dst1)

        exdrain(c2 - 2, ex0, x0)
        idwait(c2, src0, dst0)
        compute(c2, src0, dst0, ex0)
        exwrite(c2, ex0, x0)
        return carry

    lax.fori_loop(0, NCHUNK // 2, body, 0)
    exdrain(NCHUNK - 1, ex0, x0)
    exdrain(NCHUNK - 2, ex1, x1)
    plsc.subcore_barrier()
    pltpu.sync_copy(accden.at[pl.ds(sid * DEN_TILE, DEN_TILE)],
                    den_hbm.at[cid, pl.ds(sid * DEN_TILE, DEN_TILE)])


def _sc_gat_body(h_hbm, ex_hbm, src_hbm, dst_hbm, zeros_hbm, num_hbm,
                 src0, src1, dst0, dst1, rows0, rows1, ex0, ex1, acc,
                 g0, g1, s0, s1, m):
    cid = lax.axis_index("c")
    sid = lax.axis_index("s")
    wid = sid * NC + cid
    r0 = sid * ROWS_PER_TILE
    ebase = wid * EPT
    pltpu.sync_copy(zeros_hbm, acc.at[pl.ds(r0, ROWS_PER_TILE)])
    plsc.subcore_barrier()

    def mul(rows_v, ex_v):
        for g in range(K // 16):
            for e in range(16):
                row = g * 16 + e
                exrow = ex_v[row, :]
                ws = [lax.gather(
                    exrow, jnp.full((16, 1), hh, _i32),
                    lax.GatherDimensionNumbers(
                        offset_dims=(), collapsed_slice_dims=(0,),
                        start_index_map=(0,)),
                    slice_sizes=(1,),
                    mode=lax.GatherScatterMode.PROMISE_IN_BOUNDS)
                      for hh in range(HEADS)]
                for hh in range(HEADS):
                    for d2 in range(2):
                        c0 = (hh * 2 + d2) * 16
                        rows_v[row, pl.ds(c0, 16)] = (
                            rows_v[row, pl.ds(c0, 16)] * ws[hh])

    def ids(i, src_v, dst_v):
        pltpu.async_copy(src_hbm.at[pl.ds(ebase + i * K, K)], src_v, m)
        pltpu.async_copy(dst_hbm.at[pl.ds(ebase + i * K, K)], dst_v, m)

    def idwait(i, src_v, dst_v):
        pltpu.make_async_copy(src_hbm.at[pl.ds(ebase + i * K, K)], src_v,
                              m).wait()
        pltpu.make_async_copy(dst_hbm.at[pl.ds(ebase + i * K, K)], dst_v,
                              m).wait()

    def gather(i, src_v, rows_v, ex_v, gsem):
        pltpu.async_copy(h_hbm.at[src_v], rows_v, gsem)
        pltpu.async_copy(ex_hbm.at[pl.ds(ebase + i * K, K)], ex_v, gsem)

    def gwait(i, src_v, rows_v, ex_v, gsem):
        pltpu.make_async_copy(h_hbm.at[src_v], rows_v, gsem).wait()
        pltpu.make_async_copy(ex_hbm.at[pl.ds(ebase + i * K, K)], ex_v,
                              gsem).wait()

    def scat(rows_v, dst_v, ssem):
        pltpu.async_copy(rows_v, acc.at[dst_v], ssem, add=True)

    def swait(rows_v, dst_v, ssem):
        pltpu.make_async_copy(rows_v, acc.at[dst_v], ssem).wait()

    pltpu.sync_copy(src_hbm.at[pl.ds(ebase, K)], src0)
    pltpu.sync_copy(dst_hbm.at[pl.ds(ebase, K)], dst0)
    gather(0, src0, rows0, ex0, g0)
    ids(1, src1, dst1)
    gwait(0, src0, rows0, ex0, g0)
    idwait(1, src1, dst1)
    gather(1, src1, rows1, ex1, g1)
    mul(rows0, ex0)
    scat(rows0, dst0, s0)

    def chunk(c, src_c, dst_c, rows_c, ex_c, gc, sc,
              src_n, dst_n, rows_n, ex_n, gn, sn, last):
        swait(rows_n, dst_n, sn)
        if last is None:
            ids(c + 1, src_n, dst_n)
        gwait(c, src_c, rows_c, ex_c, gc)
        if last is None:
            idwait(c + 1, src_n, dst_n)
            gather(c + 1, src_n, rows_n, ex_n, gn)
        mul(rows_c, ex_c)
        scat(rows_c, dst_c, sc)

    def body(j, carry):
        c1 = 2 * j + 1
        chunk(c1, src1, dst1, rows1, ex1, g1, s1,
              src0, dst0, rows0, ex0, g0, s0, None)
        c2 = 2 * j + 2
        swait(rows1, dst1, s1)

        @pl.when(j != NCHUNK // 2 - 1)
        def _():
            ids(c2 + 1, src1, dst1)

        gwait(c2, src0, rows0, ex0, g0)

        @pl.when(j != NCHUNK // 2 - 1)
        def _():
            idwait(c2 + 1, src1, dst1)
            gather(c2 + 1, src1, rows1, ex1, g1)

        mul(rows0, ex0)
        scat(rows0, dst0, s0)
        return carry

    lax.fori_loop(0, NCHUNK // 2, body, 0)
    swait(rows0, dst0, s0)
    plsc.subcore_barrier()
    pltpu.sync_copy(acc.at[pl.ds(r0, ROWS_PER_TILE)],
                    num_hbm.at[cid, pl.ds(r0, ROWS_PER_TILE)])


def _sc_sage_body(h_hbm, src_hbm, dst_hbm, zeros_hbm, s_hbm,
                  src0, src1, dst0, dst1, rows0, rows1, acc,
                  g0, g1, s0, s1, m):
    cid = lax.axis_index("c")
    sid = lax.axis_index("s")
    wid = sid * NC + cid
    r0 = sid * ROWS_PER_TILE
    ebase = wid * EPT
    pltpu.sync_copy(zeros_hbm, acc.at[pl.ds(r0, ROWS_PER_TILE)])
    plsc.subcore_barrier()

    def ids(i, src_v, dst_v):
        pltpu.async_copy(src_hbm.at[pl.ds(ebase + i * K, K)], src_v, m)
        pltpu.async_copy(dst_hbm.at[pl.ds(ebase + i * K, K)], dst_v, m)

    def idwait(i, src_v, dst_v):
        pltpu.make_async_copy(src_hbm.at[pl.ds(ebase + i * K, K)], src_v,
                              m).wait()
        pltpu.make_async_copy(dst_hbm.at[pl.ds(ebase + i * K, K)], dst_v,
                              m).wait()

    def gather(src_v, rows_v, gsem):
        pltpu.async_copy(h_hbm.at[src_v], rows_v, gsem)

    def gwait(src_v, rows_v, gsem):
        pltpu.make_async_copy(h_hbm.at[src_v], rows_v, gsem).wait()

    def scat(rows_v, dst_v, ssem):
        pltpu.async_copy(rows_v, acc.at[dst_v], ssem, add=True)

    def swait(rows_v, dst_v, ssem):
        pltpu.make_async_copy(rows_v, acc.at[dst_v], ssem).wait()

    pltpu.sync_copy(src_hbm.at[pl.ds(ebase, K)], src0)
    pltpu.sync_copy(dst_hbm.at[pl.ds(ebase, K)], dst0)
    gather(src0, rows0, g0)
    ids(1, src1, dst1)
    gwait(src0, rows0, g0)
    idwait(1, src1, dst1)
    gather(src1, rows1, g1)
    scat(rows0, dst0, s0)

    def body(j, carry):
        c1 = 2 * j + 1
        swait(rows0, dst0, s0)
        ids(c1 + 1, src0, dst0)
        gwait(src1, rows1, g1)
        idwait(c1 + 1, src0, dst0)
        gather(src0, rows0, g0)
        scat(rows1, dst1, s1)
        c2 = 2 * j + 2
        swait(rows1, dst1, s1)

        @pl.when(j != NCHUNK // 2 - 1)
        def _():
            ids(c2 + 1, src1, dst1)

        gwait(src0, rows0, g0)

        @pl.when(j != NCHUNK // 2 - 1)
        def _():
            idwait(c2 + 1, src1, dst1)
            gather(src1, rows1, g1)

        scat(rows0, dst0, s0)
        return carry

    lax.fori_loop(0, NCHUNK // 2, body, 0)
    swait(rows0, dst0, s0)
    plsc.subcore_barrier()
    pltpu.sync_copy(acc.at[pl.ds(r0, ROWS_PER_TILE)],
                    s_hbm.at[cid, pl.ds(r0, ROWS_PER_TILE)])


def _make_sc_mesh():
    return plsc.VectorSubcoreMesh(core_axis_name="c", subcore_axis_name="s")


_SC_PARAMS = pltpu.CompilerParams(needs_layout_passes=False)


def _sc_gatex(asrc_flat, adst_flat, src, dst, zeros):
    return pl.kernel(
        _sc_gatex_body,
        out_type=[
            jax.ShapeDtypeStruct((N_EDGES, 16), _f32),
            jax.ShapeDtypeStruct((NC, DEN_ROWS, D), _f32),
        ],
        mesh=_make_sc_mesh(),
        scratch_types=[
            pltpu.VMEM((K,), _i32),
            pltpu.VMEM((K,), _i32),
            pltpu.VMEM((K,), _i32),
            pltpu.VMEM((K,), _i32),
            pltpu.VMEM((K,), _i32),
            pltpu.VMEM((K, 16), _f32),
            pltpu.VMEM((K, 16), _f32),
            pltpu.VMEM((K, D), _f32),
            pltpu.VMEM((HEADS * N_NODES,), _f32),
            pltpu.VMEM((HEADS * N_NODES,), _f32),
            pltpu.VMEM_SHARED((DEN_ROWS, D), _f32),
            pltpu.SemaphoreType.DMA,
            pltpu.SemaphoreType.DMA,
            pltpu.SemaphoreType.DMA,
        ],
        compiler_params=_SC_PARAMS,
    )(asrc_flat, adst_flat, src, dst, zeros)


def _sc_gat(h, ex, src, dst, zeros):
    return pl.kernel(
        _sc_gat_body,
        out_type=jax.ShapeDtypeStruct((NC, N_PAD, D), _f32),
        mesh=_make_sc_mesh(),
        scratch_types=[
            pltpu.VMEM((K,), _i32),
            pltpu.VMEM((K,), _i32),
            pltpu.VMEM((K,), _i32),
            pltpu.VMEM((K,), _i32),
            pltpu.VMEM((K, D), _f32),
            pltpu.VMEM((K, D), _f32),
            pltpu.VMEM((K, 16), _f32),
            pltpu.VMEM((K, 16), _f32),
            pltpu.VMEM_SHARED((N_PAD, D), _f32),
            pltpu.SemaphoreType.DMA,
            pltpu.SemaphoreType.DMA,
            pltpu.SemaphoreType.DMA,
            pltpu.SemaphoreType.DMA,
            pltpu.SemaphoreType.DMA,
        ],
        compiler_params=_SC_PARAMS,
    )(h, ex, src, dst, zeros)


def _sc_sage(h, src, dst, zeros):
    return pl.kernel(
        _sc_sage_body,
        out_type=jax.ShapeDtypeStruct((NC, N_PAD, D), _f32),
        mesh=_make_sc_mesh(),
        scratch_types=[
            pltpu.VMEM((K,), _i32),
            pltpu.VMEM((K,), _i32),
            pltpu.VMEM((K,), _i32),
            pltpu.VMEM((K,), _i32),
            pltpu.VMEM((K, D), _f32),
            pltpu.VMEM((K, D), _f32),
            pltpu.VMEM_SHARED((N_PAD, D), _f32),
            pltpu.SemaphoreType.DMA,
            pltpu.SemaphoreType.DMA,
            pltpu.SemaphoreType.DMA,
            pltpu.SemaphoreType.DMA,
            pltpu.SemaphoreType.DMA,
        ],
        compiler_params=_SC_PARAMS,
    )(h, src, dst, zeros)


_HI = dict(preferred_element_type=_f32, precision=lax.Precision.HIGHEST)
BROW = 2000
GRID = N_NODES // BROW


def _full(shape):
    return pl.BlockSpec(shape, lambda i: (0,) * len(shape))


def _rows(minor):
    return pl.BlockSpec((BROW, minor), lambda i: (i, 0))


def _prows(minor):
    return pl.BlockSpec((NC, BROW, minor), lambda i: (0, i, 0))


def _tc_pre_body(x_ref, w_ref, ms_ref, md_ref, wr_ref, br_ref,
                 h_ref, as_ref, ad_ref, res_ref):
    x = x_ref[...]
    h = jnp.dot(x, w_ref[...], **_HI)
    h_ref[...] = h
    as_ref[...] = jnp.dot(h, ms_ref[...], **_HI)
    ad_ref[...] = jnp.dot(h, md_ref[...], **_HI)
    res_ref[...] = jnp.dot(x, wr_ref[...], **_HI) + br_ref[...]


def _tc_pre(x, w, ms, md, wr, br):
    return pl.pallas_call(
        _tc_pre_body,
        grid=(GRID,),
        in_specs=[_rows(D), _full((D, D)), _full((D, HEADS)),
                  _full((D, HEADS)), _full((D, D)), _full((1, D))],
        out_specs=[_rows(D), _rows(HEADS), _rows(HEADS), _rows(D)],
        out_shape=[
            jax.ShapeDtypeStruct((N_NODES, D), _f32),
            jax.ShapeDtypeStruct((N_NODES, HEADS), _f32),
            jax.ShapeDtypeStruct((N_NODES, HEADS), _f32),
            jax.ShapeDtypeStruct((N_NODES, D), _f32),
        ],
    )(x, w, ms, md, wr, br)


def _bn_lrelu(out, g, b, res):
    m = jnp.mean(out, axis=0, keepdims=True)
    v = jnp.mean((out - m) * (out - m), axis=0, keepdims=True)
    out = (out - m) / jnp.sqrt(v + 1e-5) * g + b
    out = out + res
    return jnp.where(out > 0, out, 0.2 * out)


def _tc_bnres_body(o_ref, g_ref, bb_ref, res_ref, hf_ref):
    hf_ref[...] = _bn_lrelu(o_ref[...], g_ref[...], bb_ref[...], res_ref[...])


def _tc_bnres(out, g, bb, res):
    return pl.pallas_call(
        _tc_bnres_body,
        out_shape=jax.ShapeDtypeStruct((N_NODES, D), _f32),
    )(out, g, bb, res)


def _tc_bn_body(o_ref, g_ref, bb_ref, hf_ref):
    hf_ref[...] = _bn_lrelu(o_ref[...], g_ref[...], bb_ref[...], 0.0)


def _tc_bn(out, g, bb):
    return pl.pallas_call(
        _tc_bn_body,
        out_shape=jax.ShapeDtypeStruct((N_NODES, D), _f32),
    )(out, g, bb)


def _tc_gatcomb_body(nump, denp, h_ref, as_ref, ad_ref, e_ref, eh_ref,
                     b_ref, o_ref, den_ref):
    num = nump[0] + nump[1]
    den16 = denp[0] + denp[1]
    aself = as_ref[...] + ad_ref[...]
    ex4 = jnp.exp(jnp.where(aself > 0, aself, 0.2 * aself))
    exx = jnp.dot(ex4, eh_ref[...], **_HI)
    numt = num + h_ref[...] * exx
    denx = jnp.dot(den16, e_ref[...], **_HI) + exx
    o_ref[...] = numt / (denx + 1e-16) + b_ref[...]
    den_ref[...] = den16


def _tc_gatcomb(nump, denp, h, as4, ad4, e, eh, b):
    return pl.pallas_call(
        _tc_gatcomb_body,
        grid=(GRID,),
        in_specs=[_prows(D), _prows(16), _rows(D), _rows(HEADS), _rows(HEADS),
                  _full((16, D)), _full((HEADS, D)), _full((1, D))],
        out_specs=[_rows(D), _rows(16)],
        out_shape=[
            jax.ShapeDtypeStruct((N_NODES, D), _f32),
            jax.ShapeDtypeStruct((N_NODES, 16), _f32),
        ],
    )(nump, denp, h, as4, ad4, e, eh, b)


def _tc_sagecomb_body(sp, den_ref, e4_ref, hin_ref, wl_ref, bl_ref, wr_ref,
                      o_ref):
    s = sp[0] + sp[1]
    cntx = jnp.dot(den_ref[...], e4_ref[...], **_HI)
    mean = s / jnp.maximum(cntx, 1.0)
    o_ref[...] = (jnp.dot(mean, wl_ref[...], **_HI) + bl_ref[...]
                  + jnp.dot(hin_ref[...], wr_ref[...], **_HI))


def _tc_sagecomb(sp, den16, e4, hin, wl, bl, wr):
    return pl.pallas_call(
        _tc_sagecomb_body,
        grid=(GRID,),
        in_specs=[_prows(D), _rows(16), _full((16, D)), _rows(D),
                  _full((D, D)), _full((1, D)), _full((D, D))],
        out_specs=_rows(D),
        out_shape=jax.ShapeDtypeStruct((N_NODES, D), _f32),
    )(sp, den16, e4, hin, wl, bl, wr)


def _tc_pool_body(h_ref, batch_ref, sums_ref, cnt_ref):
    i = pl.program_id(0)
    bt = batch_ref[...]
    oh = (bt == lax.broadcasted_iota(_i32, (1, N_GRAPHS), 1)).astype(_f32)
    part = lax.dot_general(oh, h_ref[...], (((0,), (0,)), ((), ())), **_HI)
    ones = jnp.ones((BROW, N_GRAPHS), _f32)
    pcnt = lax.dot_general(oh, ones, (((0,), (0,)), ((), ())),
                           preferred_element_type=_f32)

    @pl.when(i == 0)
    def _():
        sums_ref[...] = jnp.zeros((N_GRAPHS, D), _f32)
        cnt_ref[...] = jnp.zeros((N_GRAPHS, N_GRAPHS), _f32)

    sums_ref[...] += part
    cnt_ref[...] += pcnt


def _tc_pool(h5, batch2d):
    return pl.pallas_call(
        _tc_pool_body,
        grid=(GRID,),
        in_specs=[_rows(D), _rows(1)],
        out_specs=[pl.BlockSpec((N_GRAPHS, D), lambda i: (0, 0)),
                   pl.BlockSpec((N_GRAPHS, N_GRAPHS), lambda i: (0, 0))],
        out_shape=[
            jax.ShapeDtypeStruct((N_GRAPHS, D), _f32),
            jax.ShapeDtypeStruct((N_GRAPHS, N_GRAPHS), _f32),
        ],
    )(h5, batch2d)


def _tc_fc_body(sums_ref, cnt_ref, w_ref, b_ref, out_ref):
    cnt = cnt_ref[:, :1]
    gm = sums_ref[...] / jnp.maximum(cnt, 1.0)
    out_ref[...] = jnp.dot(gm, w_ref[...], **_HI) + b_ref[...]


def _tc_fc(sums, cnt, w, b):
    return pl.pallas_call(
        _tc_fc_body,
        out_shape=jax.ShapeDtypeStruct((N_GRAPHS, w.shape[1]), _f32),
    )(sums, cnt, w, b)


def _att_mat(att):
    a = att.reshape(HEADS, CH)
    eye = jnp.eye(HEADS, dtype=_f32)
    return jnp.einsum('hc,hk->hck', a, eye).reshape(D, HEADS)


_E16 = np.zeros((16, D), np.float32)
for _h in range(HEADS):
    _E16[_h, _h * CH:(_h + 1) * CH] = 1.0

_EH = np.zeros((HEADS, D), np.float32)
for _h in range(HEADS):
    _EH[_h, _h * CH:(_h + 1) * CH] = 1.0

_E4 = np.zeros((16, D), np.float32)
_E4[HEADS, :] = 1.0


def kernel(x, edge_index, batch, params):
    p = params
    src = edge_index[0]
    dst = edge_index[1]
    zeros = jnp.zeros((ROWS_PER_TILE, D), _f32)
    e16 = jnp.asarray(_E16)
    eh = jnp.asarray(_EH)
    e4 = jnp.asarray(_E4)

    def row(v):
        return v.reshape(1, -1)

    h1, as1, ad1, res1 = _tc_pre(x, p['gat1']['W'],
                                 _att_mat(p['gat1']['att_src']),
                                 _att_mat(p['gat1']['att_dst']),
                                 p['res1']['W'], row(p['res1']['b']))
    ex1, den1 = _sc_gatex(as1.reshape(-1), ad1.reshape(-1), src, dst, zeros)
    num1 = _sc_gat(h1, ex1, src, dst, zeros)
    den1 = den1.reshape(NC, N_PAD, 16)
    o1, den16 = _tc_gatcomb(num1, den1, h1, as1, ad1, e16, eh,
                            row(p['gat1']['b']))
    h1f = _tc_bnres(o1, row(p['bn1']['g']), row(p['bn1']['b']), res1)

    h2, as2, ad2, res2 = _tc_pre(h1f, p['gat2']['W'],
                                 _att_mat(p['gat2']['att_src']),
                                 _att_mat(p['gat2']['att_dst']),
                                 p['res2']['W'], row(p['res2']['b']))
    ex2, den2 = _sc_gatex(as2.reshape(-1), ad2.reshape(-1), src, dst, zeros)
    num2 = _sc_gat(h2, ex2, src, dst, zeros)
    den2 = den2.reshape(NC, N_PAD, 16)
    o2, _ = _tc_gatcomb(num2, den2, h2, as2, ad2, e16, eh,
                        row(p['gat2']['b']))
    h2f = _tc_bnres(o2, row(p['bn2']['g']), row(p['bn2']['b']), res2)

    h = h2f
    for name, bn in (('sage3', 'bn3'), ('sage4', 'bn4'), ('sage5', 'bn5')):
        sp = _sc_sage(h, src, dst, zeros)
        o = _tc_sagecomb(sp, den16, e4, h,
                         p[name]['Wl'], row(p[name]['bl']), p[name]['Wr'])
        h = _tc_bn(o, row(p[bn]['g']), row(p[bn]['b']))

    sums, cnt = _tc_pool(h, batch.reshape(-1, 1))
    return _tc_fc(sums, cnt, p['fc']['W'], row(p['fc']['b']))


# speedup vs baseline: 54.5614x; 1.0009x over previous
import jax
import jax.numpy as jnp
import numpy as np
from jax import lax
from jax.experimental import pallas as pl
from jax.experimental.pallas import tpu as pltpu
from jax.experimental.pallas import tpu_sc as plsc

N_NODES = 10000
N_PAD = 10240
N_EDGES = 320000
D = 128
HEADS = 4
CH = 32
N_GRAPHS = 16

NC = 2
NS = 16
NW = NC * NS
K = 80
EPT = N_EDGES // NW
NCHUNK = EPT // K
ROWS_PER_TILE = N_PAD // NS

_f32 = jnp.float32
_i32 = jnp.int32


DEN_ROWS = N_PAD // 8
DEN_TILE = DEN_ROWS // NS


def _sc_gatex_body(asrc_hbm, adst_hbm, src_hbm, dst_hbm, zeros_hbm,
                   ex_hbm, den_hbm,
                   src0, src1, dst0, dst1, drow_v, ex0, ex1, exw_v,
                   asrc_t, adst_t, accden, m, x0, x1):
    cid = lax.axis_index("c")
    sid = lax.axis_index("s")
    wid = sid * NC + cid
    ebase = wid * EPT
    pltpu.sync_copy(asrc_hbm, asrc_t)
    pltpu.sync_copy(adst_hbm, adst_t)
    pltpu.sync_copy(zeros_hbm.at[pl.ds(0, DEN_TILE)],
                    accden.at[pl.ds(sid * DEN_TILE, DEN_TILE)])
    pltpu.sync_copy(zeros_hbm.at[pl.ds(0, K)], exw_v)
    plsc.subcore_barrier()

    lanes = lax.iota(_i32, 16)
    ones16 = jnp.ones((16,), _f32)
    zeros16v = jnp.zeros((16,), _f32)

    def ids(i, src_v, dst_v):
        pltpu.async_copy(src_hbm.at[pl.ds(ebase + i * K, K)], src_v, m)
        pltpu.async_copy(dst_hbm.at[pl.ds(ebase + i * K, K)], dst_v, m)

    def idwait(i, src_v, dst_v):
        pltpu.make_async_copy(src_hbm.at[pl.ds(ebase + i * K, K)], src_v,
                              m).wait()
        pltpu.make_async_copy(dst_hbm.at[pl.ds(ebase + i * K, K)], dst_v,
                              m).wait()

    def compute(i, src_v, dst_v, ex_v):
        for g in range(K // 16):
            ridx = lanes + (g * 16)
            sids = src_v[pl.ds(g * 16, 16)] * 4
            dvals = dst_v[pl.ds(g * 16, 16)]
            dids = dvals * 4
            drow_v[pl.ds(g * 16, 16)] = lax.shift_right_logical(dvals, 3)
            cbase = (dvals & 7) * 16
            for hh in range(HEADS):
                s_ = plsc.load_gather(asrc_t, [sids + hh])
                d_ = plsc.load_gather(adst_t, [dids + hh])
                al = s_ + d_
                al = jnp.where(al > 0, al, 0.2 * al)
                exv = jnp.exp(al)
                plsc.store_scatter(ex_v, [ridx, jnp.full((16,), hh, _i32)], exv)
                plsc.store_scatter(exw_v, [ridx, cbase + hh], exv)
            plsc.store_scatter(ex_v, [ridx, jnp.full((16,), HEADS, _i32)], ones16)
            plsc.store_scatter(exw_v, [ridx, cbase + HEADS], ones16)
        pltpu.sync_copy(exw_v, accden.at[drow_v], add=True)
        for g in range(K // 16):
            ridx = lanes + (g * 16)
            cbase = (dst_v[pl.ds(g * 16, 16)] & 7) * 16
            for hh in range(HEADS + 1):
                plsc.store_scatter(exw_v, [ridx, cbase + hh], zeros16v)

    def exwrite(i, ex_v, xsem):
        pltpu.async_copy(ex_v, ex_hbm.at[pl.ds(ebase + i * K, K)], xsem)

    def exdrain(i, ex_v, xsem):
        pltpu.make_async_copy(ex_v, ex_hbm.at[pl.ds(ebase + i * K, K)],
                              xsem).wait()

    pltpu.sync_copy(src_hbm.at[pl.ds(ebase, K)], src0)
    pltpu.sync_copy(dst_hbm.at[pl.ds(ebase, K)], dst0)
    ids(1, src1, dst1)
    compute(0, src0, dst0, ex0)
    exwrite(0, ex0, x0)

    def body(j, carry):
        c1 = 2 * j + 1
        ids(c1 + 1, src0, dst0)

        @pl.when(j > 0)
        def _():
            exdrain(c1 - 2, ex1, x1)

        idwait(c1, src1, dst1)
        compute(c1, src1, dst1, ex1)
        exwrite(c1, ex1, x1)

        c2 = 2 * j + 2

        @pl.when(j != NCHUNK // 2 - 1)
        def _():
            ids(c2 + 1, src1, dst1)

        exdrain(c2 - 2, ex0, x0)
        idwait(c2, src0, dst0)
        compute(c2, src0, dst0, ex0)
        exwrite(c2, ex0, x0)
        return carry

    lax.fori_loop(0, NCHUNK // 2, body, 0)
    exdrain(NCHUNK - 1, ex0, x0)
    exdrain(NCHUNK - 2, ex1, x1)
    plsc.subcore_barrier()
    pltpu.sync_copy(accden.at[pl.ds(sid * DEN_TILE, DEN_TILE)],
                    den_hbm.at[cid, pl.ds(sid * DEN_TILE, DEN_TILE)])


def _sc_gat_body(h_hbm, ex_hbm, src_hbm, dst_hbm, zeros_hbm, num_hbm,
                 src0, src1, dst0, dst1, rows0, rows1, ex0, ex1, acc,
                 g0, g1, s0, s1, m):
    cid = lax.axis_index("c")
    sid = lax.axis_index("s")
    wid = sid * NC + cid
    r0 = sid * ROWS_PER_TILE
    ebase = wid * EPT
    pltpu.sync_copy(zeros_hbm, acc.at[pl.ds(r0, ROWS_PER_TILE)])
    plsc.subcore_barrier()

    def mul(rows_v, ex_v):
        for g in range(K // 16):
            for e in range(16):
                row = g * 16 + e
                exrow = ex_v[row, :]
                ws = [lax.gather(
                    exrow, jnp.full((16, 1), hh, _i32),
                    lax.GatherDimensionNumbers(
                        offset_dims=(), collapsed_slice_dims=(0,),
                        start_index_map=(0,)),
                    slice_sizes=(1,),
                    mode=lax.GatherScatterMode.PROMISE_IN_BOUNDS)
                      for hh in range(HEADS)]
                for hh in range(HEADS):
                    for d2 in range(2):
                        c0 = (hh * 2 + d2) * 16
                        rows_v[row, pl.ds(c0, 16)] = (
                            rows_v[row, pl.ds(c0, 16)] * ws[hh])

    def ids(i, src_v, dst_v):
        pltpu.async_copy(src_hbm.at[pl.ds(ebase + i * K, K)], src_v, m)
        pltpu.async_copy(dst_hbm.at[pl.ds(ebase + i * K, K)], dst_v, m)

    def idwait(i, src_v, dst_v):
        pltpu.make_async_copy(src_hbm.at[pl.ds(ebase + i * K, K)], src_v,
                              m).wait()
        pltpu.make_async_copy(dst_hbm.at[pl.ds(ebase + i * K, K)], dst_v,
                              m).wait()

    def gather(i, src_v, rows_v, ex_v, gsem):
        pltpu.async_copy(h_hbm.at[src_v], rows_v, gsem)
        pltpu.async_copy(ex_hbm.at[pl.ds(ebase + i * K, K)], ex_v, gsem)

    def gwait(i, src_v, rows_v, ex_v, gsem):
        pltpu.make_async_copy(h_hbm.at[src_v], rows_v, gsem).wait()
        pltpu.make_async_copy(ex_hbm.at[pl.ds(ebase + i * K, K)], ex_v,
                              gsem).wait()

    def scat(rows_v, dst_v, ssem):
        pltpu.async_copy(rows_v, acc.at[dst_v], ssem, add=True)

    def swait(rows_v, dst_v, ssem):
        pltpu.make_async_copy(rows_v, acc.at[dst_v], ssem).wait()

    pltpu.sync_copy(src_hbm.at[pl.ds(ebase, K)], src0)
    pltpu.sync_copy(dst_hbm.at[pl.ds(ebase, K)], dst0)
    gather(0, src0, rows0, ex0, g0)
    ids(1, src1, dst1)
    gwait(0, src0, rows0, ex0, g0)
    idwait(1, src1, dst1)
    gather(1, src1, rows1, ex1, g1)
    mul(rows0, ex0)
    scat(rows0, dst0, s0)

    def chunk(c, src_c, dst_c, rows_c, ex_c, gc, sc,
              src_n, dst_n, rows_n, ex_n, gn, sn, last):
        swait(rows_n, dst_n, sn)
        if last is None:
            ids(c + 1, src_n, dst_n)
        gwait(c, src_c, rows_c, ex_c, gc)
        if last is None:
            idwait(c + 1, src_n, dst_n)
            gather(c + 1, src_n, rows_n, ex_n, gn)
        mul(rows_c, ex_c)
        scat(rows_c, dst_c, sc)

    def body(j, carry):
        c1 = 2 * j + 1
        chunk(c1, src1, dst1, rows1, ex1, g1, s1,
              src0, dst0, rows0, ex0, g0, s0, None)
        c2 = 2 * j + 2
        swait(rows1, dst1, s1)

        @pl.when(j != NCHUNK // 2 - 1)
        def _():
            ids(c2 + 1, src1, dst1)

        gwait(c2, src0, rows0, ex0, g0)

        @pl.when(j != NCHUNK // 2 - 1)
        def _():
            idwait(c2 + 1, src1, dst1)
            gather(c2 + 1, src1, rows1, ex1, g1)

        mul(rows0, ex0)
        scat(rows0, dst0, s0)
        return carry

    lax.fori_loop(0, NCHUNK // 2, body, 0)
    swait(rows0, dst0, s0)
    plsc.subcore_barrier()
    pltpu.sync_copy(acc.at[pl.ds(r0, ROWS_PER_TILE)],
                    num_hbm.at[cid, pl.ds(r0, ROWS_PER_TILE)])


def _sc_sage_body(h_hbm, src_hbm, dst_hbm, zeros_hbm, s_hbm,
                  src0, src1, dst0, dst1, rows0, rows1, acc,
                  g0, g1, s0, s1, m):
    cid = lax.axis_index("c")
    sid = lax.axis_index("s")
    wid = sid * NC + cid
    r0 = sid * ROWS_PER_TILE
    ebase = wid * EPT
    pltpu.sync_copy(zeros_hbm, acc.at[pl.ds(r0, ROWS_PER_TILE)])
    plsc.subcore_barrier()

    def ids(i, src_v, dst_v):
        pltpu.async_copy(src_hbm.at[pl.ds(ebase + i * K, K)], src_v, m)
        pltpu.async_copy(dst_hbm.at[pl.ds(ebase + i * K, K)], dst_v, m)

    def idwait(i, src_v, dst_v):
        pltpu.make_async_copy(src_hbm.at[pl.ds(ebase + i * K, K)], src_v,
                              m).wait()
        pltpu.make_async_copy(dst_hbm.at[pl.ds(ebase + i * K, K)], dst_v,
                              m).wait()

    def gather(src_v, rows_v, gsem):
        pltpu.async_copy(h_hbm.at[src_v], rows_v, gsem)

    def gwait(src_v, rows_v, gsem):
        pltpu.make_async_copy(h_hbm.at[src_v], rows_v, gsem).wait()

    def scat(rows_v, dst_v, ssem):
        pltpu.async_copy(rows_v, acc.at[dst_v], ssem, add=True)

    def swait(rows_v, dst_v, ssem):
        pltpu.make_async_copy(rows_v, acc.at[dst_v], ssem).wait()

    pltpu.sync_copy(src_hbm.at[pl.ds(ebase, K)], src0)
    pltpu.sync_copy(dst_hbm.at[pl.ds(ebase, K)], dst0)
    gather(src0, rows0, g0)
    ids(1, src1, dst1)
    gwait(src0, rows0, g0)
    idwait(1, src1, dst1)
    gather(src1, rows1, g1)
    scat(rows0, dst0, s0)

    def body(j, carry):
        c1 = 2 * j + 1
        swait(rows0, dst0, s0)
        ids(c1 + 1, src0, dst0)
        gwait(src1, rows1, g1)
        idwait(c1 + 1, src0, dst0)
        gather(src0, rows0, g0)
        scat(rows1, dst1, s1)
        c2 = 2 * j + 2
        swait(rows1, dst1, s1)

        @pl.when(j != NCHUNK // 2 - 1)
        def _():
            ids(c2 + 1, src1, dst1)

        gwait(src0, rows0, g0)

        @pl.when(j != NCHUNK // 2 - 1)
        def _():
            idwait(c2 + 1, src1, dst1)
            gather(src1, rows1, g1)

        scat(rows0, dst0, s0)
        return carry

    lax.fori_loop(0, NCHUNK // 2, body, 0)
    swait(rows0, dst0, s0)
    plsc.subcore_barrier()
    pltpu.sync_copy(acc.at[pl.ds(r0, ROWS_PER_TILE)],
                    s_hbm.at[cid, pl.ds(r0, ROWS_PER_TILE)])


def _make_sc_mesh():
    return plsc.VectorSubcoreMesh(core_axis_name="c", subcore_axis_name="s")


_SC_PARAMS = pltpu.CompilerParams(needs_layout_passes=False)


def _sc_gatex(asrc_flat, adst_flat, src, dst, zeros):
    return pl.kernel(
        _sc_gatex_body,
        out_type=[
            jax.ShapeDtypeStruct((N_EDGES, 16), _f32),
            jax.ShapeDtypeStruct((NC, DEN_ROWS, D), _f32),
        ],
        mesh=_make_sc_mesh(),
        scratch_types=[
            pltpu.VMEM((K,), _i32),
            pltpu.VMEM((K,), _i32),
            pltpu.VMEM((K,), _i32),
            pltpu.VMEM((K,), _i32),
            pltpu.VMEM((K,), _i32),
            pltpu.VMEM((K, 16), _f32),
            pltpu.VMEM((K, 16), _f32),
            pltpu.VMEM((K, D), _f32),
            pltpu.VMEM((HEADS * N_NODES,), _f32),
            pltpu.VMEM((HEADS * N_NODES,), _f32),
            pltpu.VMEM_SHARED((DEN_ROWS, D), _f32),
            pltpu.SemaphoreType.DMA,
            pltpu.SemaphoreType.DMA,
            pltpu.SemaphoreType.DMA,
        ],
        compiler_params=_SC_PARAMS,
    )(asrc_flat, adst_flat, src, dst, zeros)


def _sc_gat(h, ex, src, dst, zeros):
    return pl.kernel(
        _sc_gat_body,
        out_type=jax.ShapeDtypeStruct((NC, N_PAD, D), _f32),
        mesh=_make_sc_mesh(),
        scratch_types=[
            pltpu.VMEM((K,), _i32),
            pltpu.VMEM((K,), _i32),
            pltpu.VMEM((K,), _i32),
            pltpu.VMEM((K,), _i32),
            pltpu.VMEM((K, D), _f32),
            pltpu.VMEM((K, D), _f32),
            pltpu.VMEM((K, 16), _f32),
            pltpu.VMEM((K, 16), _f32),
            pltpu.VMEM_SHARED((N_PAD, D), _f32),
            pltpu.SemaphoreType.DMA,
            pltpu.SemaphoreType.DMA,
            pltpu.SemaphoreType.DMA,
            pltpu.SemaphoreType.DMA,
            pltpu.SemaphoreType.DMA,
        ],
        compiler_params=_SC_PARAMS,
    )(h, ex, src, dst, zeros)


def _sc_sage(h, src, dst, zeros):
    return pl.kernel(
        _sc_sage_body,
        out_type=jax.ShapeDtypeStruct((NC, N_PAD, D), _f32),
        mesh=_make_sc_mesh(),
        scratch_types=[
            pltpu.VMEM((K,), _i32),
            pltpu.VMEM((K,), _i32),
            pltpu.VMEM((K,), _i32),
            pltpu.VMEM((K,), _i32),
            pltpu.VMEM((K, D), _f32),
            pltpu.VMEM((K, D), _f32),
            pltpu.VMEM_SHARED((N_PAD, D), _f32),
            pltpu.SemaphoreType.DMA,
            pltpu.SemaphoreType.DMA,
            pltpu.SemaphoreType.DMA,
            pltpu.SemaphoreType.DMA,
            pltpu.SemaphoreType.DMA,
        ],
        compiler_params=_SC_PARAMS,
    )(h, src, dst, zeros)


_HI = dict(preferred_element_type=_f32, precision=lax.Precision.HIGHEST)
BROW = 2000
GRID = N_NODES // BROW


def _full(shape):
    return pl.BlockSpec(shape, lambda i: (0,) * len(shape))


def _rows(minor):
    return pl.BlockSpec((BROW, minor), lambda i: (i, 0))


def _prows(minor):
    return pl.BlockSpec((NC, BROW, minor), lambda i: (0, i, 0))


def _tc_pre_body(x_ref, w_ref, ms_ref, md_ref, wr_ref, br_ref,
                 h_ref, as_ref, ad_ref, res_ref):
    x = x_ref[...]
    h = jnp.dot(x, w_ref[...], **_HI)
    h_ref[...] = h
    as_ref[...] = jnp.dot(h, ms_ref[...], **_HI)
    ad_ref[...] = jnp.dot(h, md_ref[...], **_HI)
    res_ref[...] = jnp.dot(x, wr_ref[...], **_HI) + br_ref[...]


def _tc_pre(x, w, ms, md, wr, br):
    return pl.pallas_call(
        _tc_pre_body,
        grid=(GRID,),
        in_specs=[_rows(D), _full((D, D)), _full((D, HEADS)),
                  _full((D, HEADS)), _full((D, D)), _full((1, D))],
        out_specs=[_rows(D), _rows(HEADS), _rows(HEADS), _rows(D)],
        out_shape=[
            jax.ShapeDtypeStruct((N_NODES, D), _f32),
            jax.ShapeDtypeStruct((N_NODES, HEADS), _f32),
            jax.ShapeDtypeStruct((N_NODES, HEADS), _f32),
            jax.ShapeDtypeStruct((N_NODES, D), _f32),
        ],
    )(x, w, ms, md, wr, br)


def _bn_lrelu(out, g, b, res):
    m = jnp.mean(out, axis=0, keepdims=True)
    v = jnp.mean((out - m) * (out - m), axis=0, keepdims=True)
    out = (out - m) / jnp.sqrt(v + 1e-5) * g + b
    out = out + res
    return jnp.where(out > 0, out, 0.2 * out)


def _tc_bnres_body(o_ref, g_ref, bb_ref, res_ref, hf_ref):
    hf_ref[...] = _bn_lrelu(o_ref[...], g_ref[...], bb_ref[...], res_ref[...])


def _tc_bnres(out, g, bb, res):
    return pl.pallas_call(
        _tc_bnres_body,
        out_shape=jax.ShapeDtypeStruct((N_NODES, D), _f32),
    )(out, g, bb, res)


def _tc_bn_body(o_ref, g_ref, bb_ref, hf_ref):
    hf_ref[...] = _bn_lrelu(o_ref[...], g_ref[...], bb_ref[...], 0.0)


def _tc_bn(out, g, bb):
    return pl.pallas_call(
        _tc_bn_body,
        out_shape=jax.ShapeDtypeStruct((N_NODES, D), _f32),
    )(out, g, bb)


def _tc_gatcomb_body(nump, denp, h_ref, as_ref, ad_ref, e_ref, eh_ref,
                     b_ref, o_ref, den_ref):
    num = nump[0] + nump[1]
    den16 = denp[0] + denp[1]
    aself = as_ref[...] + ad_ref[...]
    ex4 = jnp.exp(jnp.where(aself > 0, aself, 0.2 * aself))
    exx = jnp.dot(ex4, eh_ref[...], **_HI)
    numt = num + h_ref[...] * exx
    denx = jnp.dot(den16, e_ref[...], **_HI) + exx
    o_ref[...] = numt / (denx + 1e-16) + b_ref[...]
    den_ref[...] = den16


def _tc_gatcomb(nump, denp, h, as4, ad4, e, eh, b):
    return pl.pallas_call(
        _tc_gatcomb_body,
        grid=(GRID,),
        in_specs=[_prows(D), _prows(16), _rows(D), _rows(HEADS), _rows(HEADS),
                  _full((16, D)), _full((HEADS, D)), _full((1, D))],
        out_specs=[_rows(D), _rows(16)],
        out_shape=[
            jax.ShapeDtypeStruct((N_NODES, D), _f32),
            jax.ShapeDtypeStruct((N_NODES, 16), _f32),
        ],
    )(nump, denp, h, as4, ad4, e, eh, b)


def _tc_sagecomb_body(sp, den_ref, e4_ref, hin_ref, wl_ref, bl_ref, wr_ref,
                      o_ref):
    s = sp[0] + sp[1]
    cntx = jnp.dot(den_ref[...], e4_ref[...], **_HI)
    mean = s / jnp.maximum(cntx, 1.0)
    o_ref[...] = (jnp.dot(mean, wl_ref[...], **_HI) + bl_ref[...]
                  + jnp.dot(hin_ref[...], wr_ref[...], **_HI))


def _tc_sagecomb(sp, den16, e4, hin, wl, bl, wr):
    return pl.pallas_call(
        _tc_sagecomb_body,
        grid=(GRID,),
        in_specs=[_prows(D), _rows(16), _full((16, D)), _rows(D),
                  _full((D, D)), _full((1, D)), _full((D, D))],
        out_specs=_rows(D),
        out_shape=jax.ShapeDtypeStruct((N_NODES, D), _f32),
    )(sp, den16, e4, hin, wl, bl, wr)


def _tc_pool_body(h_ref, batch_ref, sums_ref, cnt_ref):
    i = pl.program_id(0)
    bt = batch_ref[...]
    oh = (bt == lax.broadcasted_iota(_i32, (1, N_GRAPHS), 1)).astype(_f32)
    part = lax.dot_general(oh, h_ref[...], (((0,), (0,)), ((), ())), **_HI)
    ones = jnp.ones((BROW, N_GRAPHS), _f32)
    pcnt = lax.dot_general(oh, ones, (((0,), (0,)), ((), ())),
                           preferred_element_type=_f32)

    @pl.when(i == 0)
    def _():
        sums_ref[...] = jnp.zeros((N_GRAPHS, D), _f32)
        cnt_ref[...] = jnp.zeros((N_GRAPHS, N_GRAPHS), _f32)

    sums_ref[...] += part
    cnt_ref[...] += pcnt


def _tc_pool(h5, batch2d):
    return pl.pallas_call(
        _tc_pool_body,
        grid=(GRID,),
        in_specs=[_rows(D), _rows(1)],
        out_specs=[pl.BlockSpec((N_GRAPHS, D), lambda i: (0, 0)),
                   pl.BlockSpec((N_GRAPHS, N_GRAPHS), lambda i: (0, 0))],
        out_shape=[
            jax.ShapeDtypeStruct((N_GRAPHS, D), _f32),
            jax.ShapeDtypeStruct((N_GRAPHS, N_GRAPHS), _f32),
        ],
    )(h5, batch2d)


def _tc_fc_body(sums_ref, cnt_ref, w_ref, b_ref, out_ref):
    cnt = cnt_ref[:, :1]
    gm = sums_ref[...] / jnp.maximum(cnt, 1.0)
    out_ref[...] = jnp.dot(gm, w_ref[...], **_HI) + b_ref[...]


def _tc_fc(sums, cnt, w, b):
    return pl.pallas_call(
        _tc_fc_body,
        out_shape=jax.ShapeDtypeStruct((N_GRAPHS, w.shape[1]), _f32),
    )(sums, cnt, w, b)


def _att_mat(att):
    a = att.reshape(HEADS, CH)
    eye = jnp.eye(HEADS, dtype=_f32)
    return jnp.einsum('hc,hk->hck', a, eye).reshape(D, HEADS)


_E16 = np.zeros((16, D), np.float32)
for _h in range(HEADS):
    _E16[_h, _h * CH:(_h + 1) * CH] = 1.0

_EH = np.zeros((HEADS, D), np.float32)
for _h in range(HEADS):
    _EH[_h, _h * CH:(_h + 1) * CH] = 1.0

_E4 = np.zeros((16, D), np.float32)
_E4[HEADS, :] = 1.0


def kernel(x, edge_index, batch, params):
    p = params
    src = edge_index[0]
    dst = edge_index[1]
    zeros = jnp.zeros((ROWS_PER_TILE, D), _f32)
    e16 = jnp.asarray(_E16)
    eh = jnp.asarray(_EH)
    e4 = jnp.asarray(_E4)

    def row(v):
        return v.reshape(1, -1)

    h1, as1, ad1, res1 = _tc_pre(x, p['gat1']['W'],
                                 _att_mat(p['gat1']['att_src']),
                                 _att_mat(p['gat1']['att_dst']),
                                 p['res1']['W'], row(p['res1']['b']))
    ex1, den1 = _sc_gatex(as1.reshape(-1), ad1.reshape(-1), src, dst, zeros)
    num1 = _sc_gat(h1, ex1, src, dst, zeros)
    den1 = den1.reshape(NC, N_PAD, 16)
    o1, den16 = _tc_gatcomb(num1, den1, h1, as1, ad1, e16, eh,
                            row(p['gat1']['b']))
    h1f = _tc_bnres(o1, row(p['bn1']['g']), row(p['bn1']['b']), res1)

    h2, as2, ad2, res2 = _tc_pre(h1f, p['gat2']['W'],
                                 _att_mat(p['gat2']['att_src']),
                                 _att_mat(p['gat2']['att_dst']),
                                 p['res2']['W'], row(p['res2']['b']))
    ex2, den2 = _sc_gatex(as2.reshape(-1), ad2.reshape(-1), src, dst, zeros)
    num2 = _sc_gat(h2, ex2, src, dst, zeros)
    den2 = den2.reshape(NC, N_PAD, 16)
    o2, _ = _tc_gatcomb(num2, den2, h2, as2, ad2, e16, eh,
                        row(p['gat2']['b']))
    h2f = _tc_bnres(o2, row(p['bn2']['g']), row(p['bn2']['b']), res2)

    h = h2f
    for name, bn in (('sage3', 'bn3'), ('sage4', 'bn4'), ('sage5', 'bn5')):
        sp = _sc_sage(h, src, dst, zeros)
        o = _tc_sagecomb(sp, den16, e4, h,
                         p[name]['Wl'], row(p[name]['bl']), p[name]['Wr'])
        h = _tc_bn(o, row(p[bn]['g']), row(p[bn]['b']))

    sums, cnt = _tc_pool(h, batch.reshape(-1, 1))
    return _tc_fc(sums, cnt, p['fc']['W'], row(p['fc']['b']))
